# Initial kernel scaffold; baseline (speedup 1.0000x reference)
import functools

import jax
import jax.numpy as jnp
from jax import lax
from jax.experimental import pallas as pl
from jax.experimental.pallas import tpu as pltpu
from jax.experimental.pallas import tpu_sc as plsc

CAP = 100000
D = 128
BN = 1024
KN = 512
MOM = 0.5


def _make_sc_gather2(n_idx):
  info = plsc.get_sparse_core_info()
  nw = info.num_cores * info.num_subcores
  n_per = n_idx // nw
  assert n_per * nw == n_idx
  ch = min(128, n_per)
  n_full = n_per // ch
  tail = n_per - n_full * ch

  mesh = plsc.VectorSubcoreMesh(core_axis_name="c", subcore_axis_name="s")
  scratch = [
      pltpu.VMEM((ch,), jnp.int32),
      pltpu.VMEM((ch, D), jnp.float32),
      pltpu.VMEM((ch, D), jnp.float32),
      pltpu.SemaphoreType.DMA,
      pltpu.SemaphoreType.DMA,
  ]
  if tail:
    scratch.append(pltpu.VMEM((tail,), jnp.int32))

  @functools.partial(
      pl.kernel,
      mesh=mesh,
      out_type=(
          jax.ShapeDtypeStruct((n_idx, D), jnp.float32),
          jax.ShapeDtypeStruct((n_idx, D), jnp.float32),
      ),
      scratch_types=tuple(scratch),
  )
  def gk(tab_s, tab_t, idxr, out_s, out_t, idx_v, rows_s, rows_t, sem_s,
         sem_t, *tail_scratch):
    wid = lax.axis_index("s") * info.num_cores + lax.axis_index("c")
    base0 = wid * n_per

    def do_chunk(base, iv, rs, rt, sz):
      base = pl.multiple_of(base, 8)
      pltpu.sync_copy(idxr.at[pl.ds(base, sz)], iv)
      c1 = pltpu.async_copy(tab_s.at[iv], rs, sem_s)
      c2 = pltpu.async_copy(tab_t.at[iv], rt, sem_t)
      c1.wait()
      c2.wait()
      pltpu.sync_copy(rs, out_s.at[pl.ds(base, sz)])
      pltpu.sync_copy(rt, out_t.at[pl.ds(base, sz)])

    def body(c, carry):
      do_chunk(base0 + c * ch, idx_v, rows_s, rows_t, ch)
      return carry

    lax.fori_loop(0, n_full, body, 0)
    if tail:
      do_chunk(base0 + n_full * ch, tail_scratch[0],
               rows_s.at[pl.ds(0, tail)], rows_t.at[pl.ds(0, tail)], tail)

  return gk


def _links_body(ic_ref, ir_ref, pred_ref, fin_ref):
  ic = ic_ref[...]
  ir = ir_ref[...]
  eq = ic == ir
  jj = lax.broadcasted_iota(jnp.int32, (BN, BN), 1)
  ii = lax.broadcasted_iota(jnp.int32, (BN, BN), 0)
  pred_ref[...] = jnp.max(jnp.where(eq & (jj < ii), jj, -1), axis=1,
                          keepdims=True)
  fin_ref[...] = jnp.max(jnp.where(eq & (jj >= ii), jj, -1), axis=1,
                         keepdims=True)


def _links(idx):
  return pl.pallas_call(
      _links_body,
      out_shape=(
          jax.ShapeDtypeStruct((BN, 1), jnp.int32),
          jax.ShapeDtypeStruct((BN, 1), jnp.int32),
      ),
  )(idx.reshape(BN, 1), idx.reshape(1, BN))


def _chain_body(pred_ref, fin_ref, old_s_ref, old_t_ref, f_s_ref, f_t_ref,
                updf_s_ref, updf_t_ref, upd_s, upd_t):
  def step(i, carry):
    p = pred_ref[i, 0]
    pm = jnp.maximum(p, 0)
    first = p < 0

    prev_s = jnp.where(first, old_s_ref[pl.ds(i, 1), :], upd_s[pl.ds(pm, 1), :])
    v_s = MOM * prev_s + (1.0 - MOM) * f_s_ref[pl.ds(i, 1), :]
    r_s = v_s * lax.rsqrt(jnp.sum(v_s * v_s))
    upd_s[pl.ds(i, 1), :] = r_s

    prev_t = jnp.where(first, old_t_ref[pl.ds(i, 1), :], upd_t[pl.ds(pm, 1), :])
    v_t = MOM * prev_t + (1.0 - MOM) * f_t_ref[pl.ds(i, 1), :]
    r_t = v_t * lax.rsqrt(jnp.sum(v_t * v_t))
    upd_t[pl.ds(i, 1), :] = r_t
    return carry

  lax.fori_loop(0, BN, step, 0)

  jj = lax.broadcasted_iota(jnp.int32, (BN, BN), 1)
  sel = (jj == fin_ref[...]).astype(jnp.float32)
  updf_s_ref[...] = jnp.dot(sel, upd_s[...],
                            preferred_element_type=jnp.float32)
  updf_t_ref[...] = jnp.dot(sel, upd_t[...],
                            preferred_element_type=jnp.float32)


def _chain(pred, fin, old_s, old_t, f_s, f_t):
  vmem = pl.BlockSpec(memory_space=pltpu.VMEM)
  return pl.pallas_call(
      _chain_body,
      in_specs=[
          pl.BlockSpec(memory_space=pltpu.SMEM),
          vmem, vmem, vmem, vmem, vmem,
      ],
      out_shape=(
          jax.ShapeDtypeStruct((BN, D), jnp.float32),
          jax.ShapeDtypeStruct((BN, D), jnp.float32),
      ),
      scratch_shapes=[
          pltpu.VMEM((BN, D), jnp.float32),
          pltpu.VMEM((BN, D), jnp.float32),
      ],
  )(pred, fin, old_s, old_t, f_s, f_t)


def _scatter_body(idx_sref, mem_ref, upd_blk, out_blk):
  del idx_sref, mem_ref
  out_blk[...] = upd_blk[...]


def _scatter(idx, mem, updf):
  grid_spec = pltpu.PrefetchScalarGridSpec(
      num_scalar_prefetch=1,
      grid=(BN,),
      in_specs=[
          pl.BlockSpec(memory_space=pl.ANY),
          pl.BlockSpec((1, D), lambda i, idx_ref: (i, 0)),
      ],
      out_specs=pl.BlockSpec((1, D), lambda i, idx_ref: (idx_ref[i], 0)),
  )
  return pl.pallas_call(
      _scatter_body,
      grid_spec=grid_spec,
      out_shape=jax.ShapeDtypeStruct((CAP, D), jnp.float32),
      input_output_aliases={1: 0},
  )(idx, mem, updf)


def kernel(f_s, f_t, s_layer, t_layer, idx, contrast_idx, memory_v1,
           memory_v2):
  idx = idx.astype(jnp.int32)
  mem_s = lax.dynamic_index_in_dim(memory_v1, s_layer, 0, keepdims=False)
  mem_t = lax.dynamic_index_in_dim(memory_v2, t_layer, 0, keepdims=False)
  full_idx = jnp.concatenate(
      [idx[:, None], contrast_idx.astype(jnp.int32)], axis=1).reshape(-1)

  g_small = _make_sc_gather2(BN)
  old_s, old_t = g_small(mem_s, mem_t, idx)

  pred, fin = _links(idx)
  updf_s, updf_t = _chain(pred, fin, old_s, old_t, f_s, f_t)

  pmem_s = _scatter(idx, mem_s, updf_s)
  pmem_t = _scatter(idx, mem_t, updf_t)

  g_big = _make_sc_gather2(BN * (KN + 1))
  w_s, w_t = g_big(pmem_s, pmem_t, full_idx)
  return w_s.reshape(BN, KN + 1, D), w_t.reshape(BN, KN + 1, D)


# speedup vs baseline: 6.4883x; 6.4883x over previous
import functools

import jax
import jax.numpy as jnp
from jax import lax
from jax.experimental import pallas as pl
from jax.experimental.pallas import tpu as pltpu
from jax.experimental.pallas import tpu_sc as plsc

CAP = 100000
D = 128
BN = 1024
KN = 512
MOM = 0.5


def _make_sc_gather2(n_idx):
  info = plsc.get_sparse_core_info()
  nw = info.num_cores * info.num_subcores
  n_per = n_idx // nw
  assert n_per * nw == n_idx
  ch = min(128, n_per)
  n_full = n_per // ch
  tail = n_per - n_full * ch

  mesh = plsc.VectorSubcoreMesh(core_axis_name="c", subcore_axis_name="s")
  scratch = [
      pltpu.VMEM((ch,), jnp.int32),
      pltpu.VMEM((ch, D), jnp.float32),
      pltpu.VMEM((ch, D), jnp.float32),
      pltpu.SemaphoreType.DMA,
      pltpu.SemaphoreType.DMA,
  ]
  if tail:
    scratch.append(pltpu.VMEM((tail,), jnp.int32))

  @functools.partial(
      pl.kernel,
      mesh=mesh,
      out_type=(
          jax.ShapeDtypeStruct((n_idx, D), jnp.float32),
          jax.ShapeDtypeStruct((n_idx, D), jnp.float32),
      ),
      scratch_types=tuple(scratch),
  )
  def gk(tab_s, tab_t, idxr, out_s, out_t, idx_v, rows_s, rows_t, sem_s,
         sem_t, *tail_scratch):
    wid = lax.axis_index("s") * info.num_cores + lax.axis_index("c")
    base0 = wid * n_per

    def do_chunk(base, iv, rs, rt, sz):
      base = pl.multiple_of(base, 8)
      pltpu.sync_copy(idxr.at[pl.ds(base, sz)], iv)
      c1 = pltpu.async_copy(tab_s.at[iv], rs, sem_s)
      c2 = pltpu.async_copy(tab_t.at[iv], rt, sem_t)
      c1.wait()
      c2.wait()
      pltpu.sync_copy(rs, out_s.at[pl.ds(base, sz)])
      pltpu.sync_copy(rt, out_t.at[pl.ds(base, sz)])

    def body(c, carry):
      do_chunk(base0 + c * ch, idx_v, rows_s, rows_t, ch)
      return carry

    lax.fori_loop(0, n_full, body, 0)
    if tail:
      do_chunk(base0 + n_full * ch, tail_scratch[0],
               rows_s.at[pl.ds(0, tail)], rows_t.at[pl.ds(0, tail)], tail)

  return gk


def _links_body(ic_ref, ir_ref, pred_ref, fin_ref):
  ic = ic_ref[...]
  ir = ir_ref[...]
  eq = ic == ir
  jj = lax.broadcasted_iota(jnp.int32, (BN, BN), 1)
  ii = lax.broadcasted_iota(jnp.int32, (BN, BN), 0)
  pred_ref[...] = jnp.max(jnp.where(eq & (jj < ii), jj, -1), axis=1,
                          keepdims=True)
  fin_ref[...] = jnp.max(jnp.where(eq & (jj >= ii), jj, -1), axis=1,
                         keepdims=True)


def _links(idx):
  return pl.pallas_call(
      _links_body,
      out_shape=(
          jax.ShapeDtypeStruct((BN, 1), jnp.int32),
          jax.ShapeDtypeStruct((BN, 1), jnp.int32),
      ),
  )(idx.reshape(BN, 1), idx.reshape(1, BN))


def _chain_body(pred_ref, fin_ref, old_s_ref, old_t_ref, f_s_ref, f_t_ref,
                updf_s_ref, updf_t_ref, upd_s, upd_t):
  def step(i, carry):
    p = pred_ref[i, 0]
    pm = jnp.maximum(p, 0)
    first = p < 0

    prev_s = jnp.where(first, old_s_ref[pl.ds(i, 1), :], upd_s[pl.ds(pm, 1), :])
    v_s = MOM * prev_s + (1.0 - MOM) * f_s_ref[pl.ds(i, 1), :]
    r_s = v_s * lax.rsqrt(jnp.sum(v_s * v_s))
    upd_s[pl.ds(i, 1), :] = r_s

    prev_t = jnp.where(first, old_t_ref[pl.ds(i, 1), :], upd_t[pl.ds(pm, 1), :])
    v_t = MOM * prev_t + (1.0 - MOM) * f_t_ref[pl.ds(i, 1), :]
    r_t = v_t * lax.rsqrt(jnp.sum(v_t * v_t))
    upd_t[pl.ds(i, 1), :] = r_t
    return carry

  lax.fori_loop(0, BN, step, 0)

  jj = lax.broadcasted_iota(jnp.int32, (BN, BN), 1)
  sel = (jj == fin_ref[...]).astype(jnp.float32)
  updf_s_ref[...] = jnp.dot(sel, upd_s[...],
                            preferred_element_type=jnp.float32)
  updf_t_ref[...] = jnp.dot(sel, upd_t[...],
                            preferred_element_type=jnp.float32)


def _chain(pred, fin, old_s, old_t, f_s, f_t):
  vmem = pl.BlockSpec(memory_space=pltpu.VMEM)
  return pl.pallas_call(
      _chain_body,
      in_specs=[
          pl.BlockSpec(memory_space=pltpu.SMEM),
          vmem, vmem, vmem, vmem, vmem,
      ],
      out_shape=(
          jax.ShapeDtypeStruct((BN, D), jnp.float32),
          jax.ShapeDtypeStruct((BN, D), jnp.float32),
      ),
      scratch_shapes=[
          pltpu.VMEM((BN, D), jnp.float32),
          pltpu.VMEM((BN, D), jnp.float32),
      ],
  )(pred, fin, old_s, old_t, f_s, f_t)


def _scatter_body(idx_sref, mem_ref, upd_blk, out_blk):
  del idx_sref, mem_ref
  out_blk[...] = upd_blk[...]


def _scatter(idx, mem, updf):
  grid_spec = pltpu.PrefetchScalarGridSpec(
      num_scalar_prefetch=1,
      grid=(BN,),
      in_specs=[
          pl.BlockSpec(memory_space=pl.ANY),
          pl.BlockSpec((1, 1, D), lambda i, idx_ref: (i, 0, 0)),
      ],
      out_specs=pl.BlockSpec((1, 1, D), lambda i, idx_ref: (idx_ref[i], 0, 0)),
  )
  out = pl.pallas_call(
      _scatter_body,
      grid_spec=grid_spec,
      out_shape=jax.ShapeDtypeStruct((CAP, 1, D), jnp.float32),
      input_output_aliases={1: 0},
  )(idx, mem.reshape(CAP, 1, D), updf.reshape(BN, 1, D))
  return out.reshape(CAP, D)


def kernel(f_s, f_t, s_layer, t_layer, idx, contrast_idx, memory_v1,
           memory_v2):
  idx = idx.astype(jnp.int32)
  mem_s = lax.dynamic_index_in_dim(memory_v1, s_layer, 0, keepdims=False)
  mem_t = lax.dynamic_index_in_dim(memory_v2, t_layer, 0, keepdims=False)
  full_idx = jnp.concatenate(
      [idx[:, None], contrast_idx.astype(jnp.int32)], axis=1).reshape(-1)

  g_small = _make_sc_gather2(BN)
  old_s, old_t = g_small(mem_s, mem_t, idx)

  pred, fin = _links(idx)
  updf_s, updf_t = _chain(pred, fin, old_s, old_t, f_s, f_t)

  pmem_s = _scatter(idx, mem_s, updf_s)
  pmem_t = _scatter(idx, mem_t, updf_t)

  g_big = _make_sc_gather2(BN * (KN + 1))
  w_s, w_t = g_big(pmem_s, pmem_t, full_idx)
  return w_s.reshape(BN, KN + 1, D), w_t.reshape(BN, KN + 1, D)


# speedup vs baseline: 9.2373x; 1.4237x over previous
import functools

import jax
import jax.numpy as jnp
from jax import lax
from jax.experimental import pallas as pl
from jax.experimental.pallas import tpu as pltpu
from jax.experimental.pallas import tpu_sc as plsc

CAP = 100000
D = 128
BN = 1024
KN = 512
MOM = 0.5

_COPY_ROWS = 800


def _make_sc_gather2(n_idx, rows_a, rows_b):
  info = plsc.get_sparse_core_info()
  nw = info.num_cores * info.num_subcores
  n_per = n_idx // nw
  assert n_per * nw == n_idx
  ch = min(128, n_per)
  n_full = n_per // ch
  tail = n_per - n_full * ch

  mesh = plsc.VectorSubcoreMesh(core_axis_name="c", subcore_axis_name="s")
  scratch = [
      pltpu.VMEM((ch,), jnp.int32),
      pltpu.VMEM((ch,), jnp.int32),
      pltpu.VMEM((ch, D), jnp.float32),
      pltpu.VMEM((ch, D), jnp.float32),
      pltpu.SemaphoreType.DMA,
      pltpu.SemaphoreType.DMA,
  ]
  if tail:
    scratch.append(pltpu.VMEM((tail,), jnp.int32))
    scratch.append(pltpu.VMEM((tail,), jnp.int32))

  @functools.partial(
      pl.kernel,
      mesh=mesh,
      out_type=(
          jax.ShapeDtypeStruct((n_idx, D), jnp.float32),
          jax.ShapeDtypeStruct((n_idx, D), jnp.float32),
      ),
      scratch_types=tuple(scratch),
  )
  def gk(tab_a, tab_b, idxr_a, idxr_b, out_a, out_b, iva, ivb, rows_va,
         rows_vb, sem_a, sem_b, *tail_scratch):
    wid = lax.axis_index("s") * info.num_cores + lax.axis_index("c")
    base0 = wid * n_per

    def do_chunk(base, ia, ib, ra, rb, sz):
      base = pl.multiple_of(base, 8)
      pltpu.sync_copy(idxr_a.at[pl.ds(base, sz)], ia)
      pltpu.sync_copy(idxr_b.at[pl.ds(base, sz)], ib)
      c1 = pltpu.async_copy(tab_a.at[ia], ra, sem_a)
      c2 = pltpu.async_copy(tab_b.at[ib], rb, sem_b)
      c1.wait()
      c2.wait()
      pltpu.sync_copy(ra, out_a.at[pl.ds(base, sz)])
      pltpu.sync_copy(rb, out_b.at[pl.ds(base, sz)])

    def body(c, carry):
      do_chunk(base0 + c * ch, iva, ivb, rows_va, rows_vb, ch)
      return carry

    lax.fori_loop(0, n_full, body, 0)
    if tail:
      do_chunk(base0 + n_full * ch, tail_scratch[0], tail_scratch[1],
               rows_va.at[pl.ds(0, tail)], rows_vb.at[pl.ds(0, tail)], tail)

  return gk


def _links_body(ic_ref, ir_ref, pred_ref, fin_ref):
  ic = ic_ref[...]
  ir = ir_ref[...]
  eq = ic == ir
  jj = lax.broadcasted_iota(jnp.int32, (BN, BN), 1)
  ii = lax.broadcasted_iota(jnp.int32, (BN, BN), 0)
  pred_ref[...] = jnp.max(jnp.where(eq & (jj < ii), jj, -1), axis=1,
                          keepdims=True)
  fin_ref[...] = jnp.max(jnp.where(eq & (jj >= ii), jj, -1), axis=1,
                         keepdims=True)


def _links(idx):
  return pl.pallas_call(
      _links_body,
      out_shape=(
          jax.ShapeDtypeStruct((BN, 1), jnp.int32),
          jax.ShapeDtypeStruct((BN, 1), jnp.int32),
      ),
  )(idx.reshape(BN, 1), idx.reshape(1, BN))


def _chain_body(pred_ref, fin_ref, old_s_ref, old_t_ref, f_s_ref, f_t_ref,
                updf_s_ref, updf_t_ref, upd_s, upd_t):
  def step(i, carry):
    p = pred_ref[i, 0]
    pm = jnp.maximum(p, 0)
    first = p < 0

    prev_s = jnp.where(first, old_s_ref[pl.ds(i, 1), :], upd_s[pl.ds(pm, 1), :])
    v_s = MOM * prev_s + (1.0 - MOM) * f_s_ref[pl.ds(i, 1), :]
    r_s = v_s * lax.rsqrt(jnp.sum(v_s * v_s))
    upd_s[pl.ds(i, 1), :] = r_s

    prev_t = jnp.where(first, old_t_ref[pl.ds(i, 1), :], upd_t[pl.ds(pm, 1), :])
    v_t = MOM * prev_t + (1.0 - MOM) * f_t_ref[pl.ds(i, 1), :]
    r_t = v_t * lax.rsqrt(jnp.sum(v_t * v_t))
    upd_t[pl.ds(i, 1), :] = r_t
    return carry

  lax.fori_loop(0, BN, step, 0)

  jj = lax.broadcasted_iota(jnp.int32, (BN, BN), 1)
  sel = (jj == fin_ref[...]).astype(jnp.float32)
  updf_s_ref[...] = jnp.dot(sel, upd_s[...],
                            preferred_element_type=jnp.float32)
  updf_t_ref[...] = jnp.dot(sel, upd_t[...],
                            preferred_element_type=jnp.float32)


def _chain(pred, fin, old_s, old_t, f_s, f_t):
  vmem = pl.BlockSpec(memory_space=pltpu.VMEM)
  return pl.pallas_call(
      _chain_body,
      in_specs=[
          pl.BlockSpec(memory_space=pltpu.SMEM),
          vmem, vmem, vmem, vmem, vmem,
      ],
      out_shape=(
          jax.ShapeDtypeStruct((BN, D), jnp.float32),
          jax.ShapeDtypeStruct((BN, D), jnp.float32),
      ),
      scratch_shapes=[
          pltpu.VMEM((BN, D), jnp.float32),
          pltpu.VMEM((BN, D), jnp.float32),
      ],
  )(pred, fin, old_s, old_t, f_s, f_t)


def _copy_body(lay_ref, in1_blk, in2_blk, out1_blk, out2_blk):
  del lay_ref
  out1_blk[...] = in1_blk[0]
  out2_blk[...] = in2_blk[0]


def _copy_layers(memory_v1, memory_v2, s_layer, t_layer):
  lays = jnp.stack([jnp.asarray(s_layer, jnp.int32),
                    jnp.asarray(t_layer, jnp.int32)])
  grid_spec = pltpu.PrefetchScalarGridSpec(
      num_scalar_prefetch=1,
      grid=(CAP // _COPY_ROWS,),
      in_specs=[
          pl.BlockSpec((1, _COPY_ROWS, D), lambda i, lay: (lay[0], i, 0)),
          pl.BlockSpec((1, _COPY_ROWS, D), lambda i, lay: (lay[1], i, 0)),
      ],
      out_specs=[
          pl.BlockSpec((_COPY_ROWS, D), lambda i, lay: (i, 0)),
          pl.BlockSpec((_COPY_ROWS, D), lambda i, lay: (i, 0)),
      ],
  )
  return pl.pallas_call(
      _copy_body,
      grid_spec=grid_spec,
      out_shape=(
          jax.ShapeDtypeStruct((CAP, D), jnp.float32),
          jax.ShapeDtypeStruct((CAP, D), jnp.float32),
      ),
  )(lays, memory_v1, memory_v2)


_RING = 16


def _patch_body(idx_ref, mem_s_ref, mem_t_ref, upd_s_ref, upd_t_ref,
                out_s_ref, out_t_ref, sem_s, sem_t):
  del mem_s_ref, mem_t_ref

  def fire(i):
    r = idx_ref[i]
    pltpu.make_async_copy(upd_s_ref.at[pl.ds(i, 1), :],
                          out_s_ref.at[pl.ds(r, 1), :], sem_s).start()
    pltpu.make_async_copy(upd_t_ref.at[pl.ds(i, 1), :],
                          out_t_ref.at[pl.ds(r, 1), :], sem_t).start()

  def drain():
    pltpu.make_async_copy(upd_s_ref.at[pl.ds(0, 1), :],
                          out_s_ref.at[pl.ds(0, 1), :], sem_s).wait()
    pltpu.make_async_copy(upd_t_ref.at[pl.ds(0, 1), :],
                          out_t_ref.at[pl.ds(0, 1), :], sem_t).wait()

  def step(i, carry):
    fire(i)

    @pl.when(i >= _RING)
    def _():
      drain()

    return carry

  lax.fori_loop(0, BN, step, 0)

  def fin_step(i, carry):
    drain()
    return carry

  lax.fori_loop(0, _RING, fin_step, 0)


def _patch(idx, pmem_s, pmem_t, updf_s, updf_t):
  vmem = pl.BlockSpec(memory_space=pltpu.VMEM)
  anyspace = pl.BlockSpec(memory_space=pl.ANY)
  return pl.pallas_call(
      _patch_body,
      in_specs=[
          pl.BlockSpec(memory_space=pltpu.SMEM),
          anyspace, anyspace, vmem, vmem,
      ],
      out_shape=(
          jax.ShapeDtypeStruct((CAP, D), jnp.float32),
          jax.ShapeDtypeStruct((CAP, D), jnp.float32),
      ),
      out_specs=[anyspace, anyspace],
      scratch_shapes=[pltpu.SemaphoreType.DMA, pltpu.SemaphoreType.DMA],
      input_output_aliases={1: 0, 2: 1},
  )(idx, pmem_s, pmem_t, updf_s, updf_t)


def kernel(f_s, f_t, s_layer, t_layer, idx, contrast_idx, memory_v1,
           memory_v2):
  idx = idx.astype(jnp.int32)
  soff = jnp.asarray(s_layer, jnp.int32) * CAP
  toff = jnp.asarray(t_layer, jnp.int32) * CAP
  mem1f = memory_v1.reshape(4 * CAP, D)
  mem2f = memory_v2.reshape(4 * CAP, D)
  full_idx = jnp.concatenate(
      [idx[:, None], contrast_idx.astype(jnp.int32)], axis=1).reshape(-1)

  g_small = _make_sc_gather2(BN, 4 * CAP, 4 * CAP)
  old_s, old_t = g_small(mem1f, mem2f, idx + soff, idx + toff)

  pred, fin = _links(idx)
  updf_s, updf_t = _chain(pred, fin, old_s, old_t, f_s, f_t)

  cmem_s, cmem_t = _copy_layers(memory_v1, memory_v2, s_layer, t_layer)
  pmem_s, pmem_t = _patch(idx, cmem_s, cmem_t, updf_s, updf_t)

  g_big = _make_sc_gather2(BN * (KN + 1), CAP, CAP)
  w_s, w_t = g_big(pmem_s, pmem_t, full_idx, full_idx)
  return w_s.reshape(BN, KN + 1, D), w_t.reshape(BN, KN + 1, D)


# speedup vs baseline: 10.7653x; 1.1654x over previous
import functools

import jax
import jax.numpy as jnp
from jax import lax
from jax.experimental import pallas as pl
from jax.experimental.pallas import tpu as pltpu
from jax.experimental.pallas import tpu_sc as plsc

CAP = 100000
D = 128
BN = 1024
KN = 512
MOM = 0.5

_COPY_ROWS = 800


def _make_sc_gather2(n_idx, rows_a, rows_b):
  info = plsc.get_sparse_core_info()
  nw = info.num_cores * info.num_subcores
  n_per = n_idx // nw
  assert n_per * nw == n_idx
  ch = min(128, n_per)
  n_full = n_per // ch
  tail = n_per - n_full * ch

  mesh = plsc.VectorSubcoreMesh(core_axis_name="c", subcore_axis_name="s")
  scratch = [
      pltpu.VMEM((ch,), jnp.int32),
      pltpu.VMEM((ch,), jnp.int32),
      pltpu.VMEM((ch, D), jnp.float32),
      pltpu.VMEM((ch, D), jnp.float32),
      pltpu.SemaphoreType.DMA,
      pltpu.SemaphoreType.DMA,
  ]
  if tail:
    scratch.append(pltpu.VMEM((tail,), jnp.int32))
    scratch.append(pltpu.VMEM((tail,), jnp.int32))

  @functools.partial(
      pl.kernel,
      mesh=mesh,
      out_type=(
          jax.ShapeDtypeStruct((n_idx, D), jnp.float32),
          jax.ShapeDtypeStruct((n_idx, D), jnp.float32),
      ),
      scratch_types=tuple(scratch),
  )
  def gk(tab_a, tab_b, idxr_a, idxr_b, out_a, out_b, iva, ivb, rows_va,
         rows_vb, sem_a, sem_b, *tail_scratch):
    wid = lax.axis_index("s") * info.num_cores + lax.axis_index("c")
    base0 = wid * n_per

    def do_chunk(base, ia, ib, ra, rb, sz):
      base = pl.multiple_of(base, 8)
      pltpu.sync_copy(idxr_a.at[pl.ds(base, sz)], ia)
      pltpu.sync_copy(idxr_b.at[pl.ds(base, sz)], ib)
      c1 = pltpu.async_copy(tab_a.at[ia], ra, sem_a)
      c2 = pltpu.async_copy(tab_b.at[ib], rb, sem_b)
      c1.wait()
      c2.wait()
      pltpu.sync_copy(ra, out_a.at[pl.ds(base, sz)])
      pltpu.sync_copy(rb, out_b.at[pl.ds(base, sz)])

    def body(c, carry):
      do_chunk(base0 + c * ch, iva, ivb, rows_va, rows_vb, ch)
      return carry

    lax.fori_loop(0, n_full, body, 0)
    if tail:
      do_chunk(base0 + n_full * ch, tail_scratch[0], tail_scratch[1],
               rows_va.at[pl.ds(0, tail)], rows_vb.at[pl.ds(0, tail)], tail)

  return gk


def _chain_body(ic_ref, ir_ref, old_s_ref, old_t_ref, f_s_ref, f_t_ref,
                updf_s_ref, updf_t_ref):
  ic = ic_ref[...]
  ir = ir_ref[...]
  eq = ic == ir
  jj = lax.broadcasted_iota(jnp.int32, (BN, BN), 1)
  ii = lax.broadcasted_iota(jnp.int32, (BN, BN), 0)
  eqlt = eq & (jj < ii)
  pred = jnp.max(jnp.where(eqlt, jj, -1), axis=1, keepdims=True)
  depth = jnp.sum(eqlt.astype(jnp.int32), axis=1, keepdims=True)
  fin = jnp.max(jnp.where(eq & (jj >= ii), jj, -1), axis=1, keepdims=True)
  psel = (jj == pred).astype(jnp.float32)
  fsel = (jj == fin).astype(jnp.float32)
  maxd = jnp.max(depth)

  old_s = old_s_ref[...]
  old_t = old_t_ref[...]
  f_s = f_s_ref[...]
  f_t = f_t_ref[...]

  def one_bank(d, upd, old, f):
    prev = jnp.where(depth == 0, old,
                     jnp.dot(psel, upd, preferred_element_type=jnp.float32))
    v = MOM * prev + (1.0 - MOM) * f
    r = v * lax.rsqrt(jnp.sum(v * v, axis=1, keepdims=True))
    return jnp.where(depth == d, r, upd)

  def cond(carry):
    return carry[0] <= maxd

  def body(carry):
    d, us, ut = carry
    return d + 1, one_bank(d, us, old_s, f_s), one_bank(d, ut, old_t, f_t)

  zeros = jnp.zeros((BN, D), jnp.float32)
  _, upd_s, upd_t = lax.while_loop(cond, body, (0, zeros, zeros))

  updf_s_ref[...] = jnp.dot(fsel, upd_s, preferred_element_type=jnp.float32)
  updf_t_ref[...] = jnp.dot(fsel, upd_t, preferred_element_type=jnp.float32)


def _chain(idx, old_s, old_t, f_s, f_t):
  return pl.pallas_call(
      _chain_body,
      out_shape=(
          jax.ShapeDtypeStruct((BN, D), jnp.float32),
          jax.ShapeDtypeStruct((BN, D), jnp.float32),
      ),
  )(idx.reshape(BN, 1), idx.reshape(1, BN), old_s, old_t, f_s, f_t)


def _copy_body(lay_ref, in1_blk, in2_blk, out1_blk, out2_blk):
  del lay_ref
  out1_blk[...] = in1_blk[0]
  out2_blk[...] = in2_blk[0]


def _copy_layers(memory_v1, memory_v2, s_layer, t_layer):
  lays = jnp.stack([jnp.asarray(s_layer, jnp.int32),
                    jnp.asarray(t_layer, jnp.int32)])
  grid_spec = pltpu.PrefetchScalarGridSpec(
      num_scalar_prefetch=1,
      grid=(CAP // _COPY_ROWS,),
      in_specs=[
          pl.BlockSpec((1, _COPY_ROWS, D), lambda i, lay: (lay[0], i, 0)),
          pl.BlockSpec((1, _COPY_ROWS, D), lambda i, lay: (lay[1], i, 0)),
      ],
      out_specs=[
          pl.BlockSpec((_COPY_ROWS, D), lambda i, lay: (i, 0)),
          pl.BlockSpec((_COPY_ROWS, D), lambda i, lay: (i, 0)),
      ],
  )
  return pl.pallas_call(
      _copy_body,
      grid_spec=grid_spec,
      out_shape=(
          jax.ShapeDtypeStruct((CAP, D), jnp.float32),
          jax.ShapeDtypeStruct((CAP, D), jnp.float32),
      ),
  )(lays, memory_v1, memory_v2)


_RING = 16


def _patch_body(idx_ref, mem_s_ref, mem_t_ref, upd_s_ref, upd_t_ref,
                out_s_ref, out_t_ref, sem_s, sem_t):
  del mem_s_ref, mem_t_ref

  def fire(i):
    r = idx_ref[i]
    pltpu.make_async_copy(upd_s_ref.at[pl.ds(i, 1), :],
                          out_s_ref.at[pl.ds(r, 1), :], sem_s).start()
    pltpu.make_async_copy(upd_t_ref.at[pl.ds(i, 1), :],
                          out_t_ref.at[pl.ds(r, 1), :], sem_t).start()

  def drain():
    pltpu.make_async_copy(upd_s_ref.at[pl.ds(0, 1), :],
                          out_s_ref.at[pl.ds(0, 1), :], sem_s).wait()
    pltpu.make_async_copy(upd_t_ref.at[pl.ds(0, 1), :],
                          out_t_ref.at[pl.ds(0, 1), :], sem_t).wait()

  def step(i, carry):
    fire(i)

    @pl.when(i >= _RING)
    def _():
      drain()

    return carry

  lax.fori_loop(0, BN, step, 0)

  def fin_step(i, carry):
    drain()
    return carry

  lax.fori_loop(0, _RING, fin_step, 0)


def _patch(idx, pmem_s, pmem_t, updf_s, updf_t):
  vmem = pl.BlockSpec(memory_space=pltpu.VMEM)
  anyspace = pl.BlockSpec(memory_space=pl.ANY)
  return pl.pallas_call(
      _patch_body,
      in_specs=[
          pl.BlockSpec(memory_space=pltpu.SMEM),
          anyspace, anyspace, vmem, vmem,
      ],
      out_shape=(
          jax.ShapeDtypeStruct((CAP, D), jnp.float32),
          jax.ShapeDtypeStruct((CAP, D), jnp.float32),
      ),
      out_specs=[anyspace, anyspace],
      scratch_shapes=[pltpu.SemaphoreType.DMA, pltpu.SemaphoreType.DMA],
      input_output_aliases={1: 0, 2: 1},
  )(idx, pmem_s, pmem_t, updf_s, updf_t)


def kernel(f_s, f_t, s_layer, t_layer, idx, contrast_idx, memory_v1,
           memory_v2):
  idx = idx.astype(jnp.int32)
  soff = jnp.asarray(s_layer, jnp.int32) * CAP
  toff = jnp.asarray(t_layer, jnp.int32) * CAP
  mem1f = memory_v1.reshape(4 * CAP, D)
  mem2f = memory_v2.reshape(4 * CAP, D)
  kp = KN + 8
  fi2 = jnp.concatenate([idx[:, None], contrast_idx.astype(jnp.int32)],
                        axis=1)
  fidx = jnp.pad(fi2, ((0, 0), (0, kp - (KN + 1)))).reshape(-1)

  g_small = _make_sc_gather2(BN, 4 * CAP, 4 * CAP)
  old_s, old_t = g_small(mem1f, mem2f, idx + soff, idx + toff)

  updf_s, updf_t = _chain(idx, old_s, old_t, f_s, f_t)

  cmem_s, cmem_t = _copy_layers(memory_v1, memory_v2, s_layer, t_layer)
  pmem_s, pmem_t = _patch(idx, cmem_s, cmem_t, updf_s, updf_t)

  g_big = _make_sc_gather2(BN * kp, CAP, CAP)
  w_s, w_t = g_big(pmem_s, pmem_t, fidx, fidx)
  return (w_s.reshape(BN, kp, D)[:, :KN + 1, :],
          w_t.reshape(BN, kp, D)[:, :KN + 1, :])


# speedup vs baseline: 16.8645x; 1.5666x over previous
import functools

import jax
import jax.numpy as jnp
from jax import lax
from jax.experimental import pallas as pl
from jax.experimental.pallas import tpu as pltpu
from jax.experimental.pallas import tpu_sc as plsc

CAP = 100000
D = 128
BN = 1024
KN = 512
MOM = 0.5

_COPY_ROWS = 800


_GROUP = 20
_NBUF = 4


def _make_sc_gather2(n_idx, ch, same_idx):
  info = plsc.get_sparse_core_info()
  nw = info.num_cores * info.num_subcores
  n_per = n_idx // nw
  assert n_per * nw == n_idx
  assert n_per % ch == 0 and ch % 8 == 0 and ch <= 128
  nchunks = n_per // ch
  pipelined = nchunks % _GROUP == 0 and nchunks >= _GROUP

  mesh = plsc.VectorSubcoreMesh(core_axis_name="c", subcore_axis_name="s")
  scratch = [pltpu.VMEM((n_per,), jnp.int32)]
  if not same_idx:
    scratch.append(pltpu.VMEM((n_per,), jnp.int32))
  nbuf = _NBUF if pipelined else 1
  scratch += [pltpu.VMEM((ch, D), jnp.float32) for _ in range(2 * nbuf)]
  scratch += [pltpu.SemaphoreType.DMA for _ in range(4 * nbuf)]

  @functools.partial(
      pl.kernel,
      mesh=mesh,
      out_type=(
          jax.ShapeDtypeStruct((n_idx, D), jnp.float32),
          jax.ShapeDtypeStruct((n_idx, D), jnp.float32),
      ),
      scratch_types=tuple(scratch),
  )
  def gk(tab_a, tab_b, idxr_a, idxr_b, out_a, out_b, *scr):
    wid = lax.axis_index("s") * info.num_cores + lax.axis_index("c")
    base0 = wid * n_per

    pos = 0
    idx_all_a = scr[pos]
    pos += 1
    if same_idx:
      idx_all_b = idx_all_a
    else:
      idx_all_b = scr[pos]
      pos += 1
    bufs_a = scr[pos:pos + nbuf]
    bufs_b = scr[pos + nbuf:pos + 2 * nbuf]
    pos += 2 * nbuf
    gsem_a = scr[pos:pos + nbuf]
    gsem_b = scr[pos + nbuf:pos + 2 * nbuf]
    ssem_a = scr[pos + 2 * nbuf:pos + 3 * nbuf]
    ssem_b = scr[pos + 3 * nbuf:pos + 4 * nbuf]

    pltpu.sync_copy(idxr_a.at[pl.ds(base0, n_per)], idx_all_a)
    if not same_idx:
      pltpu.sync_copy(idxr_b.at[pl.ds(base0, n_per)], idx_all_b)

    def start_gather(c, b):
      iva = idx_all_a.at[pl.ds(pl.multiple_of(c * ch, 8), ch)]
      ivb = idx_all_b.at[pl.ds(pl.multiple_of(c * ch, 8), ch)]
      return (pltpu.async_copy(tab_a.at[iva], bufs_a[b], gsem_a[b]),
              pltpu.async_copy(tab_b.at[ivb], bufs_b[b], gsem_b[b]))

    def start_store(c, b):
      dst = pl.multiple_of(base0 + c * ch, 8)
      return (pltpu.async_copy(bufs_a[b], out_a.at[pl.ds(dst, ch)], ssem_a[b]),
              pltpu.async_copy(bufs_b[b], out_b.at[pl.ds(dst, ch)], ssem_b[b]))

    if not pipelined:
      def body(c, carry):
        ga, gb = start_gather(c, 0)
        ga.wait()
        gb.wait()
        sa, sb = start_store(c, 0)
        sa.wait()
        sb.wait()
        return carry

      lax.fori_loop(0, nchunks, body, 0)
      return

    def group(g, carry):
      c0 = g * _GROUP
      gobjs = {}
      sobjs = {}
      gobjs[0] = start_gather(c0 + 0, 0)
      gobjs[1] = start_gather(c0 + 1, 1)
      for j in range(_GROUP):
        ga, gb = gobjs.pop(j)
        ga.wait()
        gb.wait()
        if j + 2 < _GROUP:
          if j - 2 >= 0:
            sa, sb = sobjs.pop(j - 2)
            sa.wait()
            sb.wait()
          gobjs[j + 2] = start_gather(c0 + j + 2, (j + 2) % _NBUF)
        sobjs[j] = start_store(c0 + j, j % _NBUF)
      for j in sorted(sobjs):
        sa, sb = sobjs[j]
        sa.wait()
        sb.wait()
      return carry

    lax.fori_loop(0, nchunks // _GROUP, group, 0)

  return gk


def _chain_body(ic_ref, ir_ref, old_s_ref, old_t_ref, f_s_ref, f_t_ref,
                updf_s_ref, updf_t_ref):
  ic = ic_ref[...]
  ir = ir_ref[...]
  eq = ic == ir
  jj = lax.broadcasted_iota(jnp.int32, (BN, BN), 1)
  ii = lax.broadcasted_iota(jnp.int32, (BN, BN), 0)
  eqlt = eq & (jj < ii)
  pred = jnp.max(jnp.where(eqlt, jj, -1), axis=1, keepdims=True)
  depth = jnp.sum(eqlt.astype(jnp.int32), axis=1, keepdims=True)
  fin = jnp.max(jnp.where(eq & (jj >= ii), jj, -1), axis=1, keepdims=True)
  psel = (jj == pred).astype(jnp.float32)
  fsel = (jj == fin).astype(jnp.float32)
  maxd = jnp.max(depth)

  old_s = old_s_ref[...]
  old_t = old_t_ref[...]
  f_s = f_s_ref[...]
  f_t = f_t_ref[...]

  def one_bank(d, upd, old, f):
    prev = jnp.where(depth == 0, old,
                     jnp.dot(psel, upd, preferred_element_type=jnp.float32))
    v = MOM * prev + (1.0 - MOM) * f
    r = v * lax.rsqrt(jnp.sum(v * v, axis=1, keepdims=True))
    return jnp.where(depth == d, r, upd)

  def cond(carry):
    return carry[0] <= maxd

  def body(carry):
    d, us, ut = carry
    return d + 1, one_bank(d, us, old_s, f_s), one_bank(d, ut, old_t, f_t)

  zeros = jnp.zeros((BN, D), jnp.float32)
  _, upd_s, upd_t = lax.while_loop(cond, body, (0, zeros, zeros))

  updf_s_ref[...] = jnp.dot(fsel, upd_s, preferred_element_type=jnp.float32)
  updf_t_ref[...] = jnp.dot(fsel, upd_t, preferred_element_type=jnp.float32)


def _chain(idx, old_s, old_t, f_s, f_t):
  return pl.pallas_call(
      _chain_body,
      out_shape=(
          jax.ShapeDtypeStruct((BN, D), jnp.float32),
          jax.ShapeDtypeStruct((BN, D), jnp.float32),
      ),
  )(idx.reshape(BN, 1), idx.reshape(1, BN), old_s, old_t, f_s, f_t)


def _copy_body(lay_ref, in1_blk, in2_blk, out1_blk, out2_blk):
  del lay_ref
  out1_blk[...] = in1_blk[0]
  out2_blk[...] = in2_blk[0]


def _copy_layers(memory_v1, memory_v2, s_layer, t_layer):
  lays = jnp.stack([jnp.asarray(s_layer, jnp.int32),
                    jnp.asarray(t_layer, jnp.int32)])
  grid_spec = pltpu.PrefetchScalarGridSpec(
      num_scalar_prefetch=1,
      grid=(CAP // _COPY_ROWS,),
      in_specs=[
          pl.BlockSpec((1, _COPY_ROWS, D), lambda i, lay: (lay[0], i, 0)),
          pl.BlockSpec((1, _COPY_ROWS, D), lambda i, lay: (lay[1], i, 0)),
      ],
      out_specs=[
          pl.BlockSpec((_COPY_ROWS, D), lambda i, lay: (i, 0)),
          pl.BlockSpec((_COPY_ROWS, D), lambda i, lay: (i, 0)),
      ],
  )
  return pl.pallas_call(
      _copy_body,
      grid_spec=grid_spec,
      out_shape=(
          jax.ShapeDtypeStruct((CAP, D), jnp.float32),
          jax.ShapeDtypeStruct((CAP, D), jnp.float32),
      ),
  )(lays, memory_v1, memory_v2)


_RING = 16


def _patch_body(idx_ref, mem_s_ref, mem_t_ref, upd_s_ref, upd_t_ref,
                out_s_ref, out_t_ref, sem_s, sem_t):
  del mem_s_ref, mem_t_ref

  def fire(i):
    r = idx_ref[i]
    pltpu.make_async_copy(upd_s_ref.at[pl.ds(i, 1), :],
                          out_s_ref.at[pl.ds(r, 1), :], sem_s).start()
    pltpu.make_async_copy(upd_t_ref.at[pl.ds(i, 1), :],
                          out_t_ref.at[pl.ds(r, 1), :], sem_t).start()

  def drain():
    pltpu.make_async_copy(upd_s_ref.at[pl.ds(0, 1), :],
                          out_s_ref.at[pl.ds(0, 1), :], sem_s).wait()
    pltpu.make_async_copy(upd_t_ref.at[pl.ds(0, 1), :],
                          out_t_ref.at[pl.ds(0, 1), :], sem_t).wait()

  def step(i, carry):
    fire(i)

    @pl.when(i >= _RING)
    def _():
      drain()

    return carry

  lax.fori_loop(0, BN, step, 0)

  def fin_step(i, carry):
    drain()
    return carry

  lax.fori_loop(0, _RING, fin_step, 0)


def _patch(idx, pmem_s, pmem_t, updf_s, updf_t):
  vmem = pl.BlockSpec(memory_space=pltpu.VMEM)
  anyspace = pl.BlockSpec(memory_space=pl.ANY)
  return pl.pallas_call(
      _patch_body,
      in_specs=[
          pl.BlockSpec(memory_space=pltpu.SMEM),
          anyspace, anyspace, vmem, vmem,
      ],
      out_shape=(
          jax.ShapeDtypeStruct((CAP, D), jnp.float32),
          jax.ShapeDtypeStruct((CAP, D), jnp.float32),
      ),
      out_specs=[anyspace, anyspace],
      scratch_shapes=[pltpu.SemaphoreType.DMA, pltpu.SemaphoreType.DMA],
      input_output_aliases={1: 0, 2: 1},
  )(idx, pmem_s, pmem_t, updf_s, updf_t)


def kernel(f_s, f_t, s_layer, t_layer, idx, contrast_idx, memory_v1,
           memory_v2):
  idx = idx.astype(jnp.int32)
  soff = jnp.asarray(s_layer, jnp.int32) * CAP
  toff = jnp.asarray(t_layer, jnp.int32) * CAP
  mem1f = memory_v1.reshape(4 * CAP, D)
  mem2f = memory_v2.reshape(4 * CAP, D)
  kp = KN + 8
  npad = kp - (KN + 1)
  padv = (lax.broadcasted_iota(jnp.int32, (BN, npad), 0) * npad
          + lax.broadcasted_iota(jnp.int32, (BN, npad), 1)) % CAP
  fi2 = jnp.concatenate(
      [idx[:, None], contrast_idx.astype(jnp.int32), padv], axis=1)
  fidx = fi2.reshape(-1)

  g_small = _make_sc_gather2(BN, 32, False)
  old_s, old_t = g_small(mem1f, mem2f, idx + soff, idx + toff)

  updf_s, updf_t = _chain(idx, old_s, old_t, f_s, f_t)

  cmem_s, cmem_t = _copy_layers(memory_v1, memory_v2, s_layer, t_layer)
  pmem_s, pmem_t = _patch(idx, cmem_s, cmem_t, updf_s, updf_t)

  g_big = _make_sc_gather2(BN * kp, 104, True)
  w_s, w_t = g_big(pmem_s, pmem_t, fidx, fidx)
  return (w_s.reshape(BN, kp, D)[:, :KN + 1, :],
          w_t.reshape(BN, kp, D)[:, :KN + 1, :])


# speedup vs baseline: 17.6923x; 1.0491x over previous
import functools

import jax
import jax.numpy as jnp
from jax import lax
from jax.experimental import pallas as pl
from jax.experimental.pallas import tpu as pltpu
from jax.experimental.pallas import tpu_sc as plsc

CAP = 100000
D = 128
BN = 1024
KN = 512
MOM = 0.5

_COPY_ROWS = 800


_GROUP = 20
_NBUF = 4


def _make_sc_gather2(n_idx, ch, same_idx):
  info = plsc.get_sparse_core_info()
  nw = info.num_cores * info.num_subcores
  n_per = n_idx // nw
  assert n_per * nw == n_idx
  assert n_per % ch == 0 and ch % 8 == 0 and ch <= 128
  nchunks = n_per // ch
  pipelined = nchunks % _GROUP == 0 and nchunks >= _GROUP

  mesh = plsc.VectorSubcoreMesh(core_axis_name="c", subcore_axis_name="s")
  scratch = [pltpu.VMEM((n_per,), jnp.int32)]
  if not same_idx:
    scratch.append(pltpu.VMEM((n_per,), jnp.int32))
  nbuf = _NBUF if pipelined else 1
  scratch += [pltpu.VMEM((ch, D), jnp.float32) for _ in range(2 * nbuf)]
  scratch += [pltpu.SemaphoreType.DMA for _ in range(4 * nbuf)]

  @functools.partial(
      pl.kernel,
      mesh=mesh,
      out_type=(
          jax.ShapeDtypeStruct((n_idx, D), jnp.float32),
          jax.ShapeDtypeStruct((n_idx, D), jnp.float32),
      ),
      scratch_types=tuple(scratch),
  )
  def gk(tab_a, tab_b, idxr_a, idxr_b, out_a, out_b, *scr):
    wid = lax.axis_index("s") * info.num_cores + lax.axis_index("c")
    base0 = wid * n_per

    pos = 0
    idx_all_a = scr[pos]
    pos += 1
    if same_idx:
      idx_all_b = idx_all_a
    else:
      idx_all_b = scr[pos]
      pos += 1
    bufs_a = scr[pos:pos + nbuf]
    bufs_b = scr[pos + nbuf:pos + 2 * nbuf]
    pos += 2 * nbuf
    gsem_a = scr[pos:pos + nbuf]
    gsem_b = scr[pos + nbuf:pos + 2 * nbuf]
    ssem_a = scr[pos + 2 * nbuf:pos + 3 * nbuf]
    ssem_b = scr[pos + 3 * nbuf:pos + 4 * nbuf]

    pltpu.sync_copy(idxr_a.at[pl.ds(base0, n_per)], idx_all_a)
    if not same_idx:
      pltpu.sync_copy(idxr_b.at[pl.ds(base0, n_per)], idx_all_b)

    def start_gather(c, b):
      iva = idx_all_a.at[pl.ds(pl.multiple_of(c * ch, 8), ch)]
      ivb = idx_all_b.at[pl.ds(pl.multiple_of(c * ch, 8), ch)]
      return (pltpu.async_copy(tab_a.at[iva], bufs_a[b], gsem_a[b]),
              pltpu.async_copy(tab_b.at[ivb], bufs_b[b], gsem_b[b]))

    def start_store(c, b):
      dst = pl.multiple_of(base0 + c * ch, 8)
      return (pltpu.async_copy(bufs_a[b], out_a.at[pl.ds(dst, ch)], ssem_a[b]),
              pltpu.async_copy(bufs_b[b], out_b.at[pl.ds(dst, ch)], ssem_b[b]))

    if not pipelined:
      def body(c, carry):
        ga, gb = start_gather(c, 0)
        ga.wait()
        gb.wait()
        sa, sb = start_store(c, 0)
        sa.wait()
        sb.wait()
        return carry

      lax.fori_loop(0, nchunks, body, 0)
      return

    def group(g, carry):
      c0 = g * _GROUP
      gobjs = {}
      sobjs = {}
      gobjs[0] = start_gather(c0 + 0, 0)
      gobjs[1] = start_gather(c0 + 1, 1)
      for j in range(_GROUP):
        ga, gb = gobjs.pop(j)
        ga.wait()
        gb.wait()
        if j + 2 < _GROUP:
          if j - 2 >= 0:
            sa, sb = sobjs.pop(j - 2)
            sa.wait()
            sb.wait()
          gobjs[j + 2] = start_gather(c0 + j + 2, (j + 2) % _NBUF)
        sobjs[j] = start_store(c0 + j, j % _NBUF)
      for j in sorted(sobjs):
        sa, sb = sobjs[j]
        sa.wait()
        sb.wait()
      return carry

    lax.fori_loop(0, nchunks // _GROUP, group, 0)

  return gk


def _make_sc_gather2_tiled(kp, ch):
  info = plsc.get_sparse_core_info()
  nw = info.num_cores * info.num_subcores
  n_per = BN * kp // nw
  cps = kp // ch
  assert cps * ch == kp and _GROUP % cps == 0
  nchunks = n_per // ch
  assert nchunks % _GROUP == 0
  b_per_g = _GROUP // cps
  last_sz = KN + 1 - (cps - 1) * ch

  mesh = plsc.VectorSubcoreMesh(core_axis_name="c", subcore_axis_name="s")
  scratch = [pltpu.VMEM((n_per,), jnp.int32)]
  scratch += [pltpu.VMEM((ch, D), jnp.float32) for _ in range(2 * _NBUF)]
  scratch += [pltpu.SemaphoreType.DMA for _ in range(4 * _NBUF)]

  @functools.partial(
      pl.kernel,
      mesh=mesh,
      out_type=(
          jax.ShapeDtypeStruct((BN, KN + 1, D), jnp.float32),
          jax.ShapeDtypeStruct((BN, KN + 1, D), jnp.float32),
      ),
      scratch_types=tuple(scratch),
      compiler_params=pltpu.CompilerParams(use_tc_tiling_on_sc=True),
  )
  def gk(tab_a, tab_b, idxr, out_a, out_b, *scr):
    wid = lax.axis_index("s") * info.num_cores + lax.axis_index("c")
    base0 = wid * n_per
    b0 = wid * (n_per // kp)

    idx_all = scr[0]
    bufs_a = scr[1:1 + _NBUF]
    bufs_b = scr[1 + _NBUF:1 + 2 * _NBUF]
    p = 1 + 2 * _NBUF
    gsem_a = scr[p:p + _NBUF]
    gsem_b = scr[p + _NBUF:p + 2 * _NBUF]
    ssem_a = scr[p + 2 * _NBUF:p + 3 * _NBUF]
    ssem_b = scr[p + 3 * _NBUF:p + 4 * _NBUF]

    pltpu.sync_copy(idxr.at[pl.ds(base0, n_per)], idx_all)

    def start_gather(c, b):
      iv = idx_all.at[pl.ds(pl.multiple_of(c * ch, 8), ch)]
      return (pltpu.async_copy(tab_a.at[iv], bufs_a[b], gsem_a[b]),
              pltpu.async_copy(tab_b.at[iv], bufs_b[b], gsem_b[b]))

    def group(g, carry):
      c0 = g * _GROUP

      def start_store(j, b):
        samp = b0 + g * b_per_g + (j // cps)
        k0 = (j % cps) * ch
        sz = ch if (j % cps) < cps - 1 else last_sz
        dst_a = out_a.at[samp, pl.ds(k0, sz), :]
        dst_b = out_b.at[samp, pl.ds(k0, sz), :]
        return (pltpu.async_copy(bufs_a[b].at[pl.ds(0, sz)], dst_a, ssem_a[b]),
                pltpu.async_copy(bufs_b[b].at[pl.ds(0, sz)], dst_b, ssem_b[b]))

      gobjs = {}
      sobjs = {}
      gobjs[0] = start_gather(c0 + 0, 0)
      gobjs[1] = start_gather(c0 + 1, 1)
      for j in range(_GROUP):
        ga, gb = gobjs.pop(j)
        ga.wait()
        gb.wait()
        if j + 2 < _GROUP:
          if j - 2 >= 0:
            sa, sb = sobjs.pop(j - 2)
            sa.wait()
            sb.wait()
          gobjs[j + 2] = start_gather(c0 + j + 2, (j + 2) % _NBUF)
        sobjs[j] = start_store(j, j % _NBUF)
      for j in sorted(sobjs):
        sa, sb = sobjs[j]
        sa.wait()
        sb.wait()
      return carry

    lax.fori_loop(0, nchunks // _GROUP, group, 0)

  return gk


def _chain_body(ic_ref, ir_ref, old_s_ref, old_t_ref, f_s_ref, f_t_ref,
                updf_s_ref, updf_t_ref):
  ic = ic_ref[...]
  ir = ir_ref[...]
  eq = ic == ir
  jj = lax.broadcasted_iota(jnp.int32, (BN, BN), 1)
  ii = lax.broadcasted_iota(jnp.int32, (BN, BN), 0)
  eqlt = eq & (jj < ii)
  pred = jnp.max(jnp.where(eqlt, jj, -1), axis=1, keepdims=True)
  depth = jnp.sum(eqlt.astype(jnp.int32), axis=1, keepdims=True)
  fin = jnp.max(jnp.where(eq & (jj >= ii), jj, -1), axis=1, keepdims=True)
  psel = (jj == pred).astype(jnp.float32)
  fsel = (jj == fin).astype(jnp.float32)
  maxd = jnp.max(depth)

  old_s = old_s_ref[...]
  old_t = old_t_ref[...]
  f_s = f_s_ref[...]
  f_t = f_t_ref[...]

  def one_bank(d, upd, old, f):
    prev = jnp.where(depth == 0, old,
                     jnp.dot(psel, upd, preferred_element_type=jnp.float32))
    v = MOM * prev + (1.0 - MOM) * f
    r = v * lax.rsqrt(jnp.sum(v * v, axis=1, keepdims=True))
    return jnp.where(depth == d, r, upd)

  def cond(carry):
    return carry[0] <= maxd

  def body(carry):
    d, us, ut = carry
    return d + 1, one_bank(d, us, old_s, f_s), one_bank(d, ut, old_t, f_t)

  zeros = jnp.zeros((BN, D), jnp.float32)
  _, upd_s, upd_t = lax.while_loop(cond, body, (0, zeros, zeros))

  updf_s_ref[...] = jnp.dot(fsel, upd_s, preferred_element_type=jnp.float32)
  updf_t_ref[...] = jnp.dot(fsel, upd_t, preferred_element_type=jnp.float32)


def _chain(idx, old_s, old_t, f_s, f_t):
  return pl.pallas_call(
      _chain_body,
      out_shape=(
          jax.ShapeDtypeStruct((BN, D), jnp.float32),
          jax.ShapeDtypeStruct((BN, D), jnp.float32),
      ),
  )(idx.reshape(BN, 1), idx.reshape(1, BN), old_s, old_t, f_s, f_t)


def _copy_body(lay_ref, in1_blk, in2_blk, out1_blk, out2_blk):
  del lay_ref
  out1_blk[...] = in1_blk[0]
  out2_blk[...] = in2_blk[0]


def _copy_layers(memory_v1, memory_v2, s_layer, t_layer):
  lays = jnp.stack([jnp.asarray(s_layer, jnp.int32),
                    jnp.asarray(t_layer, jnp.int32)])
  grid_spec = pltpu.PrefetchScalarGridSpec(
      num_scalar_prefetch=1,
      grid=(CAP // _COPY_ROWS,),
      in_specs=[
          pl.BlockSpec((1, _COPY_ROWS, D), lambda i, lay: (lay[0], i, 0)),
          pl.BlockSpec((1, _COPY_ROWS, D), lambda i, lay: (lay[1], i, 0)),
      ],
      out_specs=[
          pl.BlockSpec((_COPY_ROWS, D), lambda i, lay: (i, 0)),
          pl.BlockSpec((_COPY_ROWS, D), lambda i, lay: (i, 0)),
      ],
  )
  return pl.pallas_call(
      _copy_body,
      grid_spec=grid_spec,
      out_shape=(
          jax.ShapeDtypeStruct((CAP, D), jnp.float32),
          jax.ShapeDtypeStruct((CAP, D), jnp.float32),
      ),
  )(lays, memory_v1, memory_v2)


_RING = 16


def _patch_body(idx_ref, mem_s_ref, mem_t_ref, upd_s_ref, upd_t_ref,
                out_s_ref, out_t_ref, sem_s, sem_t):
  del mem_s_ref, mem_t_ref

  def fire(i):
    r = idx_ref[i]
    pltpu.make_async_copy(upd_s_ref.at[pl.ds(i, 1), :],
                          out_s_ref.at[pl.ds(r, 1), :], sem_s).start()
    pltpu.make_async_copy(upd_t_ref.at[pl.ds(i, 1), :],
                          out_t_ref.at[pl.ds(r, 1), :], sem_t).start()

  def drain():
    pltpu.make_async_copy(upd_s_ref.at[pl.ds(0, 1), :],
                          out_s_ref.at[pl.ds(0, 1), :], sem_s).wait()
    pltpu.make_async_copy(upd_t_ref.at[pl.ds(0, 1), :],
                          out_t_ref.at[pl.ds(0, 1), :], sem_t).wait()

  def step(i, carry):
    fire(i)

    @pl.when(i >= _RING)
    def _():
      drain()

    return carry

  lax.fori_loop(0, BN, step, 0)

  def fin_step(i, carry):
    drain()
    return carry

  lax.fori_loop(0, _RING, fin_step, 0)


def _patch(idx, pmem_s, pmem_t, updf_s, updf_t):
  vmem = pl.BlockSpec(memory_space=pltpu.VMEM)
  anyspace = pl.BlockSpec(memory_space=pl.ANY)
  return pl.pallas_call(
      _patch_body,
      in_specs=[
          pl.BlockSpec(memory_space=pltpu.SMEM),
          anyspace, anyspace, vmem, vmem,
      ],
      out_shape=(
          jax.ShapeDtypeStruct((CAP, D), jnp.float32),
          jax.ShapeDtypeStruct((CAP, D), jnp.float32),
      ),
      out_specs=[anyspace, anyspace],
      scratch_shapes=[pltpu.SemaphoreType.DMA, pltpu.SemaphoreType.DMA],
      input_output_aliases={1: 0, 2: 1},
  )(idx, pmem_s, pmem_t, updf_s, updf_t)


def kernel(f_s, f_t, s_layer, t_layer, idx, contrast_idx, memory_v1,
           memory_v2):
  idx = idx.astype(jnp.int32)
  soff = jnp.asarray(s_layer, jnp.int32) * CAP
  toff = jnp.asarray(t_layer, jnp.int32) * CAP
  mem1f = memory_v1.reshape(4 * CAP, D)
  mem2f = memory_v2.reshape(4 * CAP, D)
  kp = KN + 8
  npad = kp - (KN + 1)
  padv = (lax.broadcasted_iota(jnp.int32, (BN, npad), 0) * npad
          + lax.broadcasted_iota(jnp.int32, (BN, npad), 1)) % CAP
  fi2 = jnp.concatenate(
      [idx[:, None], contrast_idx.astype(jnp.int32), padv], axis=1)
  fidx = fi2.reshape(-1)

  g_small = _make_sc_gather2(BN, 32, False)
  old_s, old_t = g_small(mem1f, mem2f, idx + soff, idx + toff)

  updf_s, updf_t = _chain(idx, old_s, old_t, f_s, f_t)

  cmem_s, cmem_t = _copy_layers(memory_v1, memory_v2, s_layer, t_layer)
  pmem_s, pmem_t = _patch(idx, cmem_s, cmem_t, updf_s, updf_t)

  g_big = _make_sc_gather2_tiled(kp, 104)
  w_s, w_t = g_big(pmem_s, pmem_t, fidx)
  return w_s, w_t


# speedup vs baseline: 28.3115x; 1.6002x over previous
import functools

import jax
import jax.numpy as jnp
from jax import lax
from jax.experimental import pallas as pl
from jax.experimental.pallas import tpu as pltpu
from jax.experimental.pallas import tpu_sc as plsc

CAP = 100000
D = 128
BN = 1024
KN = 512
MOM = 0.5

_COPY_ROWS = 800


_NBUF = 4


def _make_sc_gather2(n_idx, ch, same_idx, group=20):
  _GROUP = group
  info = plsc.get_sparse_core_info()
  nw = info.num_cores * info.num_subcores
  n_per = n_idx // nw
  assert n_per * nw == n_idx
  assert n_per % ch == 0 and ch % 8 == 0 and ch <= 128
  nchunks = n_per // ch
  pipelined = nchunks % _GROUP == 0 and nchunks >= _GROUP

  mesh = plsc.VectorSubcoreMesh(core_axis_name="c", subcore_axis_name="s")
  scratch = [pltpu.VMEM((n_per,), jnp.int32)]
  if not same_idx:
    scratch.append(pltpu.VMEM((n_per,), jnp.int32))
  nbuf = _NBUF if pipelined else 1
  scratch += [pltpu.VMEM((ch, D), jnp.float32) for _ in range(2 * nbuf)]
  scratch += [pltpu.SemaphoreType.DMA for _ in range(4 * nbuf)]

  @functools.partial(
      pl.kernel,
      mesh=mesh,
      out_type=(
          jax.ShapeDtypeStruct((n_idx, D), jnp.float32),
          jax.ShapeDtypeStruct((n_idx, D), jnp.float32),
      ),
      scratch_types=tuple(scratch),
  )
  def gk(tab_a, tab_b, idxr_a, idxr_b, out_a, out_b, *scr):
    wid = lax.axis_index("s") * info.num_cores + lax.axis_index("c")
    base0 = wid * n_per

    pos = 0
    idx_all_a = scr[pos]
    pos += 1
    if same_idx:
      idx_all_b = idx_all_a
    else:
      idx_all_b = scr[pos]
      pos += 1
    bufs_a = scr[pos:pos + nbuf]
    bufs_b = scr[pos + nbuf:pos + 2 * nbuf]
    pos += 2 * nbuf
    gsem_a = scr[pos:pos + nbuf]
    gsem_b = scr[pos + nbuf:pos + 2 * nbuf]
    ssem_a = scr[pos + 2 * nbuf:pos + 3 * nbuf]
    ssem_b = scr[pos + 3 * nbuf:pos + 4 * nbuf]

    pltpu.sync_copy(idxr_a.at[pl.ds(base0, n_per)], idx_all_a)
    if not same_idx:
      pltpu.sync_copy(idxr_b.at[pl.ds(base0, n_per)], idx_all_b)

    def start_gather(c, b):
      iva = idx_all_a.at[pl.ds(pl.multiple_of(c * ch, 8), ch)]
      ivb = idx_all_b.at[pl.ds(pl.multiple_of(c * ch, 8), ch)]
      return (pltpu.async_copy(tab_a.at[iva], bufs_a[b], gsem_a[b]),
              pltpu.async_copy(tab_b.at[ivb], bufs_b[b], gsem_b[b]))

    def start_store(c, b):
      dst = pl.multiple_of(base0 + c * ch, 8)
      return (pltpu.async_copy(bufs_a[b], out_a.at[pl.ds(dst, ch)], ssem_a[b]),
              pltpu.async_copy(bufs_b[b], out_b.at[pl.ds(dst, ch)], ssem_b[b]))

    if not pipelined:
      def body(c, carry):
        ga, gb = start_gather(c, 0)
        ga.wait()
        gb.wait()
        sa, sb = start_store(c, 0)
        sa.wait()
        sb.wait()
        return carry

      lax.fori_loop(0, nchunks, body, 0)
      return

    def group(g, carry):
      c0 = g * _GROUP
      gobjs = {}
      sobjs = {}
      gobjs[0] = start_gather(c0 + 0, 0)
      gobjs[1] = start_gather(c0 + 1, 1)
      for j in range(_GROUP):
        ga, gb = gobjs.pop(j)
        ga.wait()
        gb.wait()
        if j + 2 < _GROUP:
          if j - 2 >= 0:
            sa, sb = sobjs.pop(j - 2)
            sa.wait()
            sb.wait()
          gobjs[j + 2] = start_gather(c0 + j + 2, (j + 2) % _NBUF)
        sobjs[j] = start_store(c0 + j, j % _NBUF)
      for j in sorted(sobjs):
        sa, sb = sobjs[j]
        sa.wait()
        sb.wait()
      return carry

    lax.fori_loop(0, nchunks // _GROUP, group, 0)

  return gk


def _chain_body(ic_ref, ir_ref, old_s_ref, old_t_ref, f_s_ref, f_t_ref,
                updf_s_ref, updf_t_ref):
  ic = ic_ref[...]
  ir = ir_ref[...]
  eq = ic == ir
  jj = lax.broadcasted_iota(jnp.int32, (BN, BN), 1)
  ii = lax.broadcasted_iota(jnp.int32, (BN, BN), 0)
  eqlt = eq & (jj < ii)
  pred = jnp.max(jnp.where(eqlt, jj, -1), axis=1, keepdims=True)
  depth = jnp.sum(eqlt.astype(jnp.int32), axis=1, keepdims=True)
  fin = jnp.max(jnp.where(eq & (jj >= ii), jj, -1), axis=1, keepdims=True)
  psel = (jj == pred).astype(jnp.float32)
  fsel = (jj == fin).astype(jnp.float32)
  maxd = jnp.max(depth)

  old_s = old_s_ref[...]
  old_t = old_t_ref[...]
  f_s = f_s_ref[...]
  f_t = f_t_ref[...]

  def one_bank(d, upd, old, f):
    prev = jnp.where(depth == 0, old,
                     jnp.dot(psel, upd, preferred_element_type=jnp.float32))
    v = MOM * prev + (1.0 - MOM) * f
    r = v * lax.rsqrt(jnp.sum(v * v, axis=1, keepdims=True))
    return jnp.where(depth == d, r, upd)

  def cond(carry):
    return carry[0] <= maxd

  def body(carry):
    d, us, ut = carry
    return d + 1, one_bank(d, us, old_s, f_s), one_bank(d, ut, old_t, f_t)

  zeros = jnp.zeros((BN, D), jnp.float32)
  _, upd_s, upd_t = lax.while_loop(cond, body, (0, zeros, zeros))

  updf_s_ref[...] = jnp.dot(fsel, upd_s, preferred_element_type=jnp.float32)
  updf_t_ref[...] = jnp.dot(fsel, upd_t, preferred_element_type=jnp.float32)


def _chain(idx, old_s, old_t, f_s, f_t):
  return pl.pallas_call(
      _chain_body,
      out_shape=(
          jax.ShapeDtypeStruct((BN, D), jnp.float32),
          jax.ShapeDtypeStruct((BN, D), jnp.float32),
      ),
  )(idx.reshape(BN, 1), idx.reshape(1, BN), old_s, old_t, f_s, f_t)


def _copy_body(lay_ref, in1_blk, in2_blk, out1_blk, out2_blk):
  del lay_ref
  out1_blk[...] = in1_blk[0]
  out2_blk[...] = in2_blk[0]


def _copy_layers(memory_v1, memory_v2, s_layer, t_layer):
  lays = jnp.stack([jnp.asarray(s_layer, jnp.int32),
                    jnp.asarray(t_layer, jnp.int32)])
  grid_spec = pltpu.PrefetchScalarGridSpec(
      num_scalar_prefetch=1,
      grid=(CAP // _COPY_ROWS,),
      in_specs=[
          pl.BlockSpec((1, _COPY_ROWS, D), lambda i, lay: (lay[0], i, 0)),
          pl.BlockSpec((1, _COPY_ROWS, D), lambda i, lay: (lay[1], i, 0)),
      ],
      out_specs=[
          pl.BlockSpec((_COPY_ROWS, D), lambda i, lay: (i, 0)),
          pl.BlockSpec((_COPY_ROWS, D), lambda i, lay: (i, 0)),
      ],
  )
  return pl.pallas_call(
      _copy_body,
      grid_spec=grid_spec,
      out_shape=(
          jax.ShapeDtypeStruct((CAP, D), jnp.float32),
          jax.ShapeDtypeStruct((CAP, D), jnp.float32),
      ),
  )(lays, memory_v1, memory_v2)


_RING = 16


def _patch_body(idx_ref, mem_s_ref, mem_t_ref, upd_s_ref, upd_t_ref,
                out_s_ref, out_t_ref, sem_s, sem_t):
  del mem_s_ref, mem_t_ref

  def fire(i):
    r = idx_ref[i]
    pltpu.make_async_copy(upd_s_ref.at[pl.ds(i, 1), :],
                          out_s_ref.at[pl.ds(r, 1), :], sem_s).start()
    pltpu.make_async_copy(upd_t_ref.at[pl.ds(i, 1), :],
                          out_t_ref.at[pl.ds(r, 1), :], sem_t).start()

  def drain():
    pltpu.make_async_copy(upd_s_ref.at[pl.ds(0, 1), :],
                          out_s_ref.at[pl.ds(0, 1), :], sem_s).wait()
    pltpu.make_async_copy(upd_t_ref.at[pl.ds(0, 1), :],
                          out_t_ref.at[pl.ds(0, 1), :], sem_t).wait()

  def step(i, carry):
    fire(i)

    @pl.when(i >= _RING)
    def _():
      drain()

    return carry

  lax.fori_loop(0, BN, step, 0)

  def fin_step(i, carry):
    drain()
    return carry

  lax.fori_loop(0, _RING, fin_step, 0)


def _patch(idx, pmem_s, pmem_t, updf_s, updf_t):
  vmem = pl.BlockSpec(memory_space=pltpu.VMEM)
  anyspace = pl.BlockSpec(memory_space=pl.ANY)
  return pl.pallas_call(
      _patch_body,
      in_specs=[
          pl.BlockSpec(memory_space=pltpu.SMEM),
          anyspace, anyspace, vmem, vmem,
      ],
      out_shape=(
          jax.ShapeDtypeStruct((CAP, D), jnp.float32),
          jax.ShapeDtypeStruct((CAP, D), jnp.float32),
      ),
      out_specs=[anyspace, anyspace],
      scratch_shapes=[pltpu.SemaphoreType.DMA, pltpu.SemaphoreType.DMA],
      input_output_aliases={1: 0, 2: 1},
  )(idx, pmem_s, pmem_t, updf_s, updf_t)


def kernel(f_s, f_t, s_layer, t_layer, idx, contrast_idx, memory_v1,
           memory_v2):
  idx = idx.astype(jnp.int32)
  soff = jnp.asarray(s_layer, jnp.int32) * CAP
  toff = jnp.asarray(t_layer, jnp.int32) * CAP
  mem1f = memory_v1.reshape(4 * CAP, D)
  mem2f = memory_v2.reshape(4 * CAP, D)
  fi2 = jnp.concatenate([idx[:, None], contrast_idx.astype(jnp.int32)],
                        axis=1)
  fidx = jnp.swapaxes(fi2, 0, 1).reshape(-1)

  g_small = _make_sc_gather2(BN, 32, False)
  old_s, old_t = g_small(mem1f, mem2f, idx + soff, idx + toff)

  updf_s, updf_t = _chain(idx, old_s, old_t, f_s, f_t)

  cmem_s, cmem_t = _copy_layers(memory_v1, memory_v2, s_layer, t_layer)
  pmem_s, pmem_t = _patch(idx, cmem_s, cmem_t, updf_s, updf_t)

  g_big = _make_sc_gather2((KN + 1) * BN, 96, True, group=19)
  w_s, w_t = g_big(pmem_s, pmem_t, fidx, fidx)
  return (jnp.swapaxes(w_s.reshape(KN + 1, BN, D), 0, 1),
          jnp.swapaxes(w_t.reshape(KN + 1, BN, D), 0, 1))


# speedup vs baseline: 29.2436x; 1.0329x over previous
import functools

import jax
import jax.numpy as jnp
from jax import lax
from jax.experimental import pallas as pl
from jax.experimental.pallas import tpu as pltpu
from jax.experimental.pallas import tpu_sc as plsc

CAP = 100000
D = 128
BN = 1024
KN = 512
MOM = 0.5

_COPY_ROWS = 2000


_NBUF = 4


def _make_sc_gather2(n_idx, ch, same_idx, group=20):
  _GROUP = group
  info = plsc.get_sparse_core_info()
  nw = info.num_cores * info.num_subcores
  n_per = n_idx // nw
  assert n_per * nw == n_idx
  assert n_per % ch == 0 and ch % 8 == 0 and ch <= 128
  nchunks = n_per // ch
  pipelined = nchunks % _GROUP == 0 and nchunks >= _GROUP

  mesh = plsc.VectorSubcoreMesh(core_axis_name="c", subcore_axis_name="s")
  scratch = [pltpu.VMEM((n_per,), jnp.int32)]
  if not same_idx:
    scratch.append(pltpu.VMEM((n_per,), jnp.int32))
  nbuf = _NBUF if pipelined else 1
  scratch += [pltpu.VMEM((ch, D), jnp.float32) for _ in range(2 * nbuf)]
  scratch += [pltpu.SemaphoreType.DMA for _ in range(4 * nbuf)]

  @functools.partial(
      pl.kernel,
      mesh=mesh,
      out_type=(
          jax.ShapeDtypeStruct((n_idx, D), jnp.float32),
          jax.ShapeDtypeStruct((n_idx, D), jnp.float32),
      ),
      scratch_types=tuple(scratch),
  )
  def gk(tab_a, tab_b, idxr_a, idxr_b, out_a, out_b, *scr):
    wid = lax.axis_index("s") * info.num_cores + lax.axis_index("c")
    base0 = wid * n_per

    pos = 0
    idx_all_a = scr[pos]
    pos += 1
    if same_idx:
      idx_all_b = idx_all_a
    else:
      idx_all_b = scr[pos]
      pos += 1
    bufs_a = scr[pos:pos + nbuf]
    bufs_b = scr[pos + nbuf:pos + 2 * nbuf]
    pos += 2 * nbuf
    gsem_a = scr[pos:pos + nbuf]
    gsem_b = scr[pos + nbuf:pos + 2 * nbuf]
    ssem_a = scr[pos + 2 * nbuf:pos + 3 * nbuf]
    ssem_b = scr[pos + 3 * nbuf:pos + 4 * nbuf]

    pltpu.sync_copy(idxr_a.at[pl.ds(base0, n_per)], idx_all_a)
    if not same_idx:
      pltpu.sync_copy(idxr_b.at[pl.ds(base0, n_per)], idx_all_b)

    def start_gather(c, b):
      iva = idx_all_a.at[pl.ds(pl.multiple_of(c * ch, 8), ch)]
      ivb = idx_all_b.at[pl.ds(pl.multiple_of(c * ch, 8), ch)]
      return (pltpu.async_copy(tab_a.at[iva], bufs_a[b], gsem_a[b]),
              pltpu.async_copy(tab_b.at[ivb], bufs_b[b], gsem_b[b]))

    def start_store(c, b):
      dst = pl.multiple_of(base0 + c * ch, 8)
      return (pltpu.async_copy(bufs_a[b], out_a.at[pl.ds(dst, ch)], ssem_a[b]),
              pltpu.async_copy(bufs_b[b], out_b.at[pl.ds(dst, ch)], ssem_b[b]))

    if not pipelined:
      def body(c, carry):
        ga, gb = start_gather(c, 0)
        ga.wait()
        gb.wait()
        sa, sb = start_store(c, 0)
        sa.wait()
        sb.wait()
        return carry

      lax.fori_loop(0, nchunks, body, 0)
      return

    def group(g, carry):
      c0 = g * _GROUP
      gobjs = {}
      sobjs = {}
      gobjs[0] = start_gather(c0 + 0, 0)
      gobjs[1] = start_gather(c0 + 1, 1)
      for j in range(_GROUP):
        ga, gb = gobjs.pop(j)
        ga.wait()
        gb.wait()
        if j + 2 < _GROUP:
          if j - 2 >= 0:
            sa, sb = sobjs.pop(j - 2)
            sa.wait()
            sb.wait()
          gobjs[j + 2] = start_gather(c0 + j + 2, (j + 2) % _NBUF)
        sobjs[j] = start_store(c0 + j, j % _NBUF)
      for j in sorted(sobjs):
        sa, sb = sobjs[j]
        sa.wait()
        sb.wait()
      return carry

    lax.fori_loop(0, nchunks // _GROUP, group, 0)

  return gk


def _make_sc_gather1(n_idx, ch, group):
  info = plsc.get_sparse_core_info()
  nw = info.num_cores * info.num_subcores
  n_per = n_idx // nw
  assert n_per * nw == n_idx and n_per % ch == 0
  nchunks = n_per // ch
  assert nchunks % group == 0

  mesh = plsc.VectorSubcoreMesh(core_axis_name="c", subcore_axis_name="s")
  scratch = [pltpu.VMEM((n_per,), jnp.int32)]
  scratch += [pltpu.VMEM((ch, D), jnp.float32) for _ in range(_NBUF)]
  scratch += [pltpu.SemaphoreType.DMA for _ in range(2 * _NBUF)]

  @functools.partial(
      pl.kernel,
      mesh=mesh,
      out_type=jax.ShapeDtypeStruct((n_idx, D), jnp.float32),
      scratch_types=tuple(scratch),
  )
  def gk(tab, idxr, out, *scr):
    wid = lax.axis_index("s") * info.num_cores + lax.axis_index("c")
    base0 = wid * n_per
    idx_all = scr[0]
    bufs = scr[1:1 + _NBUF]
    gsem = scr[1 + _NBUF:1 + 2 * _NBUF]
    ssem = scr[1 + 2 * _NBUF:1 + 3 * _NBUF]

    pltpu.sync_copy(idxr.at[pl.ds(base0, n_per)], idx_all)

    def start_gather(c, b):
      iv = idx_all.at[pl.ds(pl.multiple_of(c * ch, 8), ch)]
      return pltpu.async_copy(tab.at[iv], bufs[b], gsem[b])

    def start_store(c, b):
      dst = pl.multiple_of(base0 + c * ch, 8)
      return pltpu.async_copy(bufs[b], out.at[pl.ds(dst, ch)], ssem[b])

    def grp(g, carry):
      c0 = g * group
      gobjs = {0: start_gather(c0 + 0, 0), 1: start_gather(c0 + 1, 1)}
      sobjs = {}
      for j in range(group):
        gobjs.pop(j).wait()
        if j + 2 < group:
          if j - 2 >= 0:
            sobjs.pop(j - 2).wait()
          gobjs[j + 2] = start_gather(c0 + j + 2, (j + 2) % _NBUF)
        sobjs[j] = start_store(c0 + j, j % _NBUF)
      for j in sorted(sobjs):
        sobjs[j].wait()
      return carry

    lax.fori_loop(0, nchunks // group, grp, 0)

  return gk


def _chain_body(ic_ref, ir_ref, old_s_ref, old_t_ref, f_s_ref, f_t_ref,
                updf_s_ref, updf_t_ref):
  ic = ic_ref[...]
  ir = ir_ref[...]
  eq = ic == ir
  jj = lax.broadcasted_iota(jnp.int32, (BN, BN), 1)
  ii = lax.broadcasted_iota(jnp.int32, (BN, BN), 0)
  eqlt = eq & (jj < ii)
  pred = jnp.max(jnp.where(eqlt, jj, -1), axis=1, keepdims=True)
  depth = jnp.sum(eqlt.astype(jnp.int32), axis=1, keepdims=True)
  fin = jnp.max(jnp.where(eq & (jj >= ii), jj, -1), axis=1, keepdims=True)
  psel = (jj == pred).astype(jnp.float32)
  fsel = (jj == fin).astype(jnp.float32)
  maxd = jnp.max(depth)

  old_s = old_s_ref[...]
  old_t = old_t_ref[...]
  f_s = f_s_ref[...]
  f_t = f_t_ref[...]

  def one_bank(d, upd, old, f):
    prev = jnp.where(depth == 0, old,
                     jnp.dot(psel, upd, preferred_element_type=jnp.float32))
    v = MOM * prev + (1.0 - MOM) * f
    r = v * lax.rsqrt(jnp.sum(v * v, axis=1, keepdims=True))
    return jnp.where(depth == d, r, upd)

  def cond(carry):
    return carry[0] <= maxd

  def body(carry):
    d, us, ut = carry
    return d + 1, one_bank(d, us, old_s, f_s), one_bank(d, ut, old_t, f_t)

  zeros = jnp.zeros((BN, D), jnp.float32)
  _, upd_s, upd_t = lax.while_loop(cond, body, (0, zeros, zeros))

  updf_s_ref[...] = jnp.dot(fsel, upd_s, preferred_element_type=jnp.float32)
  updf_t_ref[...] = jnp.dot(fsel, upd_t, preferred_element_type=jnp.float32)


def _chain(idx, old_s, old_t, f_s, f_t):
  return pl.pallas_call(
      _chain_body,
      out_shape=(
          jax.ShapeDtypeStruct((BN, D), jnp.float32),
          jax.ShapeDtypeStruct((BN, D), jnp.float32),
      ),
  )(idx.reshape(BN, 1), idx.reshape(1, BN), old_s, old_t, f_s, f_t)


def _copy_body(lay_ref, in_blk, out_blk):
  del lay_ref
  out_blk[...] = in_blk[0]


def _copy_layer(memory, layer):
  lay = jnp.asarray(layer, jnp.int32).reshape(1)
  grid_spec = pltpu.PrefetchScalarGridSpec(
      num_scalar_prefetch=1,
      grid=(CAP // _COPY_ROWS,),
      in_specs=[
          pl.BlockSpec((1, _COPY_ROWS, D), lambda i, lay: (lay[0], i, 0)),
      ],
      out_specs=pl.BlockSpec((_COPY_ROWS, D), lambda i, lay: (i, 0)),
  )
  return pl.pallas_call(
      _copy_body,
      grid_spec=grid_spec,
      out_shape=jax.ShapeDtypeStruct((CAP, D), jnp.float32),
  )(lay, memory)


_RING = 16


def _patch_body(idx_ref, mem_ref, upd_ref, out_ref, sem):
  del mem_ref

  def drain():
    pltpu.make_async_copy(upd_ref.at[pl.ds(0, 1), :],
                          out_ref.at[pl.ds(0, 1), :], sem).wait()

  def step(i, carry):
    r = idx_ref[i]
    pltpu.make_async_copy(upd_ref.at[pl.ds(i, 1), :],
                          out_ref.at[pl.ds(r, 1), :], sem).start()

    @pl.when(i >= _RING)
    def _():
      drain()

    return carry

  lax.fori_loop(0, BN, step, 0)

  def fin_step(i, carry):
    drain()
    return carry

  lax.fori_loop(0, _RING, fin_step, 0)


def _patch(idx, pmem, updf):
  vmem = pl.BlockSpec(memory_space=pltpu.VMEM)
  anyspace = pl.BlockSpec(memory_space=pl.ANY)
  return pl.pallas_call(
      _patch_body,
      in_specs=[
          pl.BlockSpec(memory_space=pltpu.SMEM),
          anyspace, vmem,
      ],
      out_shape=jax.ShapeDtypeStruct((CAP, D), jnp.float32),
      out_specs=anyspace,
      scratch_shapes=[pltpu.SemaphoreType.DMA],
      input_output_aliases={1: 0},
  )(idx, pmem, updf)


def kernel(f_s, f_t, s_layer, t_layer, idx, contrast_idx, memory_v1,
           memory_v2):
  idx = idx.astype(jnp.int32)
  soff = jnp.asarray(s_layer, jnp.int32) * CAP
  toff = jnp.asarray(t_layer, jnp.int32) * CAP
  mem1f = memory_v1.reshape(4 * CAP, D)
  mem2f = memory_v2.reshape(4 * CAP, D)
  fi2 = jnp.concatenate([idx[:, None], contrast_idx.astype(jnp.int32)],
                        axis=1)
  fidx = jnp.swapaxes(fi2, 0, 1).reshape(-1)

  g_small = _make_sc_gather2(BN, 32, False)
  old_s, old_t = g_small(mem1f, mem2f, idx + soff, idx + toff)

  updf_s, updf_t = _chain(idx, old_s, old_t, f_s, f_t)

  g_big = _make_sc_gather1((KN + 1) * BN, 96, 19)
  pmem_s = _patch(idx, _copy_layer(memory_v1, s_layer), updf_s)
  w_s = g_big(pmem_s, fidx)
  pmem_t = _patch(idx, _copy_layer(memory_v2, t_layer), updf_t)
  w_t = g_big(pmem_t, fidx)
  return (jnp.swapaxes(w_s.reshape(KN + 1, BN, D), 0, 1),
          jnp.swapaxes(w_t.reshape(KN + 1, BN, D), 0, 1))


# speedup vs baseline: 29.7130x; 1.0161x over previous
import functools

import jax
import jax.numpy as jnp
from jax import lax
from jax.experimental import pallas as pl
from jax.experimental.pallas import tpu as pltpu
from jax.experimental.pallas import tpu_sc as plsc

CAP = 100000
D = 128
BN = 1024
KN = 512
MOM = 0.5

_COPY_ROWS = 2000


_NBUF = 4


def _make_sc_gather2(n_idx, ch, same_idx, group=20):
  _GROUP = group
  info = plsc.get_sparse_core_info()
  nw = info.num_cores * info.num_subcores
  n_per = n_idx // nw
  assert n_per * nw == n_idx
  assert n_per % ch == 0 and ch % 8 == 0 and ch <= 128
  nchunks = n_per // ch
  pipelined = nchunks % _GROUP == 0 and nchunks >= _GROUP

  mesh = plsc.VectorSubcoreMesh(core_axis_name="c", subcore_axis_name="s")
  scratch = [pltpu.VMEM((n_per,), jnp.int32)]
  if not same_idx:
    scratch.append(pltpu.VMEM((n_per,), jnp.int32))
  nbuf = _NBUF if pipelined else 1
  scratch += [pltpu.VMEM((ch, D), jnp.float32) for _ in range(2 * nbuf)]
  scratch += [pltpu.SemaphoreType.DMA for _ in range(4 * nbuf)]

  @functools.partial(
      pl.kernel,
      mesh=mesh,
      out_type=(
          jax.ShapeDtypeStruct((n_idx, D), jnp.float32),
          jax.ShapeDtypeStruct((n_idx, D), jnp.float32),
      ),
      scratch_types=tuple(scratch),
  )
  def gk(tab_a, tab_b, idxr_a, idxr_b, out_a, out_b, *scr):
    wid = lax.axis_index("s") * info.num_cores + lax.axis_index("c")
    base0 = wid * n_per

    pos = 0
    idx_all_a = scr[pos]
    pos += 1
    if same_idx:
      idx_all_b = idx_all_a
    else:
      idx_all_b = scr[pos]
      pos += 1
    bufs_a = scr[pos:pos + nbuf]
    bufs_b = scr[pos + nbuf:pos + 2 * nbuf]
    pos += 2 * nbuf
    gsem_a = scr[pos:pos + nbuf]
    gsem_b = scr[pos + nbuf:pos + 2 * nbuf]
    ssem_a = scr[pos + 2 * nbuf:pos + 3 * nbuf]
    ssem_b = scr[pos + 3 * nbuf:pos + 4 * nbuf]

    pltpu.sync_copy(idxr_a.at[pl.ds(base0, n_per)], idx_all_a)
    if not same_idx:
      pltpu.sync_copy(idxr_b.at[pl.ds(base0, n_per)], idx_all_b)

    def start_gather(c, b):
      iva = idx_all_a.at[pl.ds(pl.multiple_of(c * ch, 8), ch)]
      ivb = idx_all_b.at[pl.ds(pl.multiple_of(c * ch, 8), ch)]
      return (pltpu.async_copy(tab_a.at[iva], bufs_a[b], gsem_a[b]),
              pltpu.async_copy(tab_b.at[ivb], bufs_b[b], gsem_b[b]))

    def start_store(c, b):
      dst = pl.multiple_of(base0 + c * ch, 8)
      return (pltpu.async_copy(bufs_a[b], out_a.at[pl.ds(dst, ch)], ssem_a[b]),
              pltpu.async_copy(bufs_b[b], out_b.at[pl.ds(dst, ch)], ssem_b[b]))

    if not pipelined:
      def body(c, carry):
        ga, gb = start_gather(c, 0)
        ga.wait()
        gb.wait()
        sa, sb = start_store(c, 0)
        sa.wait()
        sb.wait()
        return carry

      lax.fori_loop(0, nchunks, body, 0)
      return

    def group(g, carry):
      c0 = g * _GROUP
      gobjs = {}
      sobjs = {}
      gobjs[0] = start_gather(c0 + 0, 0)
      gobjs[1] = start_gather(c0 + 1, 1)
      for j in range(_GROUP):
        ga, gb = gobjs.pop(j)
        ga.wait()
        gb.wait()
        if j + 2 < _GROUP:
          if j - 2 >= 0:
            sa, sb = sobjs.pop(j - 2)
            sa.wait()
            sb.wait()
          gobjs[j + 2] = start_gather(c0 + j + 2, (j + 2) % _NBUF)
        sobjs[j] = start_store(c0 + j, j % _NBUF)
      for j in sorted(sobjs):
        sa, sb = sobjs[j]
        sa.wait()
        sb.wait()
      return carry

    lax.fori_loop(0, nchunks // _GROUP, group, 0)

  return gk


def _make_sc_gather1(n_idx, ch, group, nbuf=6, ga=4):
  info = plsc.get_sparse_core_info()
  nw = info.num_cores * info.num_subcores
  n_per = n_idx // nw
  assert n_per * nw == n_idx and n_per % ch == 0
  nchunks = n_per // ch
  assert nchunks % group == 0 and ga < nbuf <= group

  mesh = plsc.VectorSubcoreMesh(core_axis_name="c", subcore_axis_name="s")
  scratch = [pltpu.VMEM((n_per,), jnp.int32)]
  scratch += [pltpu.VMEM((ch, D), jnp.float32) for _ in range(nbuf)]
  scratch += [pltpu.SemaphoreType.DMA for _ in range(2 * nbuf)]

  @functools.partial(
      pl.kernel,
      mesh=mesh,
      out_type=jax.ShapeDtypeStruct((n_idx, D), jnp.float32),
      scratch_types=tuple(scratch),
  )
  def gk(tab, idxr, out, *scr):
    wid = lax.axis_index("s") * info.num_cores + lax.axis_index("c")
    base0 = wid * n_per
    idx_all = scr[0]
    bufs = scr[1:1 + nbuf]
    gsem = scr[1 + nbuf:1 + 2 * nbuf]
    ssem = scr[1 + 2 * nbuf:1 + 3 * nbuf]

    pltpu.sync_copy(idxr.at[pl.ds(base0, n_per)], idx_all)

    def start_gather(c, b):
      iv = idx_all.at[pl.ds(pl.multiple_of(c * ch, 8), ch)]
      return pltpu.async_copy(tab.at[iv], bufs[b], gsem[b])

    def start_store(c, b):
      dst = pl.multiple_of(base0 + c * ch, 8)
      return pltpu.async_copy(bufs[b], out.at[pl.ds(dst, ch)], ssem[b])

    def grp(g, carry):
      c0 = g * group
      gobjs = {k: start_gather(c0 + k, k % nbuf) for k in range(ga)}
      sobjs = {}
      for j in range(group):
        gobjs.pop(j).wait()
        if j + ga < group:
          if j + ga - nbuf >= 0:
            sobjs.pop(j + ga - nbuf).wait()
          gobjs[j + ga] = start_gather(c0 + j + ga, (j + ga) % nbuf)
        sobjs[j] = start_store(c0 + j, j % nbuf)
      for j in sorted(sobjs):
        sobjs[j].wait()
      return carry

    lax.fori_loop(0, nchunks // group, grp, 0)

  return gk


def _chain_body(ic_ref, ir_ref, old_s_ref, old_t_ref, f_s_ref, f_t_ref,
                updf_s_ref, updf_t_ref):
  ic = ic_ref[...]
  ir = ir_ref[...]
  eq = ic == ir
  jj = lax.broadcasted_iota(jnp.int32, (BN, BN), 1)
  ii = lax.broadcasted_iota(jnp.int32, (BN, BN), 0)
  eqlt = eq & (jj < ii)
  pred = jnp.max(jnp.where(eqlt, jj, -1), axis=1, keepdims=True)
  depth = jnp.sum(eqlt.astype(jnp.int32), axis=1, keepdims=True)
  fin = jnp.max(jnp.where(eq & (jj >= ii), jj, -1), axis=1, keepdims=True)
  psel = (jj == pred).astype(jnp.float32)
  fsel = (jj == fin).astype(jnp.float32)
  maxd = jnp.max(depth)

  old_s = old_s_ref[...]
  old_t = old_t_ref[...]
  f_s = f_s_ref[...]
  f_t = f_t_ref[...]

  def one_bank(d, upd, old, f):
    prev = jnp.where(depth == 0, old,
                     jnp.dot(psel, upd, preferred_element_type=jnp.float32))
    v = MOM * prev + (1.0 - MOM) * f
    r = v * lax.rsqrt(jnp.sum(v * v, axis=1, keepdims=True))
    return jnp.where(depth == d, r, upd)

  def cond(carry):
    return carry[0] <= maxd

  def body(carry):
    d, us, ut = carry
    return d + 1, one_bank(d, us, old_s, f_s), one_bank(d, ut, old_t, f_t)

  zeros = jnp.zeros((BN, D), jnp.float32)
  _, upd_s, upd_t = lax.while_loop(cond, body, (0, zeros, zeros))

  updf_s_ref[...] = jnp.dot(fsel, upd_s, preferred_element_type=jnp.float32)
  updf_t_ref[...] = jnp.dot(fsel, upd_t, preferred_element_type=jnp.float32)


def _chain(idx, old_s, old_t, f_s, f_t):
  return pl.pallas_call(
      _chain_body,
      out_shape=(
          jax.ShapeDtypeStruct((BN, D), jnp.float32),
          jax.ShapeDtypeStruct((BN, D), jnp.float32),
      ),
  )(idx.reshape(BN, 1), idx.reshape(1, BN), old_s, old_t, f_s, f_t)


def _copy_body(lay_ref, in_blk, out_blk):
  del lay_ref
  out_blk[...] = in_blk[0]


def _copy_layer(memory, layer):
  lay = jnp.asarray(layer, jnp.int32).reshape(1)
  grid_spec = pltpu.PrefetchScalarGridSpec(
      num_scalar_prefetch=1,
      grid=(CAP // _COPY_ROWS,),
      in_specs=[
          pl.BlockSpec((1, _COPY_ROWS, D), lambda i, lay: (lay[0], i, 0)),
      ],
      out_specs=pl.BlockSpec((_COPY_ROWS, D), lambda i, lay: (i, 0)),
  )
  return pl.pallas_call(
      _copy_body,
      grid_spec=grid_spec,
      out_shape=jax.ShapeDtypeStruct((CAP, D), jnp.float32),
  )(lay, memory)


_RING = 16


def _patch_body(idx_ref, mem_ref, upd_ref, out_ref, sem):
  del mem_ref

  def drain():
    pltpu.make_async_copy(upd_ref.at[pl.ds(0, 1), :],
                          out_ref.at[pl.ds(0, 1), :], sem).wait()

  def step(i, carry):
    r = idx_ref[i]
    pltpu.make_async_copy(upd_ref.at[pl.ds(i, 1), :],
                          out_ref.at[pl.ds(r, 1), :], sem).start()

    @pl.when(i >= _RING)
    def _():
      drain()

    return carry

  lax.fori_loop(0, BN, step, 0)

  def fin_step(i, carry):
    drain()
    return carry

  lax.fori_loop(0, _RING, fin_step, 0)


def _patch(idx, pmem, updf):
  vmem = pl.BlockSpec(memory_space=pltpu.VMEM)
  anyspace = pl.BlockSpec(memory_space=pl.ANY)
  return pl.pallas_call(
      _patch_body,
      in_specs=[
          pl.BlockSpec(memory_space=pltpu.SMEM),
          anyspace, vmem,
      ],
      out_shape=jax.ShapeDtypeStruct((CAP, D), jnp.float32),
      out_specs=anyspace,
      scratch_shapes=[pltpu.SemaphoreType.DMA],
      input_output_aliases={1: 0},
  )(idx, pmem, updf)


def kernel(f_s, f_t, s_layer, t_layer, idx, contrast_idx, memory_v1,
           memory_v2):
  idx = idx.astype(jnp.int32)
  soff = jnp.asarray(s_layer, jnp.int32) * CAP
  toff = jnp.asarray(t_layer, jnp.int32) * CAP
  mem1f = memory_v1.reshape(4 * CAP, D)
  mem2f = memory_v2.reshape(4 * CAP, D)
  fi2 = jnp.concatenate([idx[:, None], contrast_idx.astype(jnp.int32)],
                        axis=1)
  fidx = jnp.swapaxes(fi2, 0, 1).reshape(-1)

  g_small = _make_sc_gather2(BN, 32, False)
  old_s, old_t = g_small(mem1f, mem2f, idx + soff, idx + toff)

  updf_s, updf_t = _chain(idx, old_s, old_t, f_s, f_t)

  g_big = _make_sc_gather1((KN + 1) * BN, 96, 19)
  pmem_s = _patch(idx, _copy_layer(memory_v1, s_layer), updf_s)
  w_s = g_big(pmem_s, fidx)
  pmem_t = _patch(idx, _copy_layer(memory_v2, t_layer), updf_t)
  w_t = g_big(pmem_t, fidx)
  return (jnp.swapaxes(w_s.reshape(KN + 1, BN, D), 0, 1),
          jnp.swapaxes(w_t.reshape(KN + 1, BN, D), 0, 1))


# speedup vs baseline: 30.4727x; 1.0256x over previous
import functools

import jax
import jax.numpy as jnp
from jax import lax
from jax.experimental import pallas as pl
from jax.experimental.pallas import tpu as pltpu
from jax.experimental.pallas import tpu_sc as plsc

CAP = 100000
D = 128
BN = 1024
KN = 512
MOM = 0.5

_COPY_ROWS = 4000


_NBUF = 4


def _make_sc_gather2(n_idx, ch, same_idx, group=20):
  _GROUP = group
  info = plsc.get_sparse_core_info()
  nw = info.num_cores * info.num_subcores
  n_per = n_idx // nw
  assert n_per * nw == n_idx
  assert n_per % ch == 0 and ch % 8 == 0 and ch <= 128
  nchunks = n_per // ch
  pipelined = nchunks % _GROUP == 0 and nchunks >= _GROUP

  mesh = plsc.VectorSubcoreMesh(core_axis_name="c", subcore_axis_name="s")
  scratch = [pltpu.VMEM((n_per,), jnp.int32)]
  if not same_idx:
    scratch.append(pltpu.VMEM((n_per,), jnp.int32))
  nbuf = _NBUF if pipelined else 1
  scratch += [pltpu.VMEM((ch, D), jnp.float32) for _ in range(2 * nbuf)]
  scratch += [pltpu.SemaphoreType.DMA for _ in range(4 * nbuf)]

  @functools.partial(
      pl.kernel,
      mesh=mesh,
      out_type=(
          jax.ShapeDtypeStruct((n_idx, D), jnp.float32),
          jax.ShapeDtypeStruct((n_idx, D), jnp.float32),
      ),
      scratch_types=tuple(scratch),
  )
  def gk(tab_a, tab_b, idxr_a, idxr_b, out_a, out_b, *scr):
    wid = lax.axis_index("s") * info.num_cores + lax.axis_index("c")
    base0 = wid * n_per

    pos = 0
    idx_all_a = scr[pos]
    pos += 1
    if same_idx:
      idx_all_b = idx_all_a
    else:
      idx_all_b = scr[pos]
      pos += 1
    bufs_a = scr[pos:pos + nbuf]
    bufs_b = scr[pos + nbuf:pos + 2 * nbuf]
    pos += 2 * nbuf
    gsem_a = scr[pos:pos + nbuf]
    gsem_b = scr[pos + nbuf:pos + 2 * nbuf]
    ssem_a = scr[pos + 2 * nbuf:pos + 3 * nbuf]
    ssem_b = scr[pos + 3 * nbuf:pos + 4 * nbuf]

    pltpu.sync_copy(idxr_a.at[pl.ds(base0, n_per)], idx_all_a)
    if not same_idx:
      pltpu.sync_copy(idxr_b.at[pl.ds(base0, n_per)], idx_all_b)

    def start_gather(c, b):
      iva = idx_all_a.at[pl.ds(pl.multiple_of(c * ch, 8), ch)]
      ivb = idx_all_b.at[pl.ds(pl.multiple_of(c * ch, 8), ch)]
      return (pltpu.async_copy(tab_a.at[iva], bufs_a[b], gsem_a[b]),
              pltpu.async_copy(tab_b.at[ivb], bufs_b[b], gsem_b[b]))

    def start_store(c, b):
      dst = pl.multiple_of(base0 + c * ch, 8)
      return (pltpu.async_copy(bufs_a[b], out_a.at[pl.ds(dst, ch)], ssem_a[b]),
              pltpu.async_copy(bufs_b[b], out_b.at[pl.ds(dst, ch)], ssem_b[b]))

    if not pipelined:
      def body(c, carry):
        ga, gb = start_gather(c, 0)
        ga.wait()
        gb.wait()
        sa, sb = start_store(c, 0)
        sa.wait()
        sb.wait()
        return carry

      lax.fori_loop(0, nchunks, body, 0)
      return

    def group(g, carry):
      c0 = g * _GROUP
      gobjs = {}
      sobjs = {}
      gobjs[0] = start_gather(c0 + 0, 0)
      gobjs[1] = start_gather(c0 + 1, 1)
      for j in range(_GROUP):
        ga, gb = gobjs.pop(j)
        ga.wait()
        gb.wait()
        if j + 2 < _GROUP:
          if j - 2 >= 0:
            sa, sb = sobjs.pop(j - 2)
            sa.wait()
            sb.wait()
          gobjs[j + 2] = start_gather(c0 + j + 2, (j + 2) % _NBUF)
        sobjs[j] = start_store(c0 + j, j % _NBUF)
      for j in sorted(sobjs):
        sa, sb = sobjs[j]
        sa.wait()
        sb.wait()
      return carry

    lax.fori_loop(0, nchunks // _GROUP, group, 0)

  return gk


def _make_sc_gather1(n_idx, ch, group, nbuf=6, ga=4):
  info = plsc.get_sparse_core_info()
  nw = info.num_cores * info.num_subcores
  n_per = n_idx // nw
  assert n_per * nw == n_idx and n_per % ch == 0
  nchunks = n_per // ch
  assert nchunks % group == 0 and ga < nbuf <= group

  mesh = plsc.VectorSubcoreMesh(core_axis_name="c", subcore_axis_name="s")
  scratch = [pltpu.VMEM((n_per,), jnp.int32)]
  scratch += [pltpu.VMEM((ch, D), jnp.float32) for _ in range(nbuf)]
  scratch += [pltpu.SemaphoreType.DMA for _ in range(2 * nbuf)]

  @functools.partial(
      pl.kernel,
      mesh=mesh,
      out_type=jax.ShapeDtypeStruct((n_idx, D), jnp.float32),
      scratch_types=tuple(scratch),
  )
  def gk(tab, idxr, out, *scr):
    wid = lax.axis_index("s") * info.num_cores + lax.axis_index("c")
    base0 = wid * n_per
    idx_all = scr[0]
    bufs = scr[1:1 + nbuf]
    gsem = scr[1 + nbuf:1 + 2 * nbuf]
    ssem = scr[1 + 2 * nbuf:1 + 3 * nbuf]

    pltpu.sync_copy(idxr.at[pl.ds(base0, n_per)], idx_all)

    def start_gather(c, b):
      iv = idx_all.at[pl.ds(pl.multiple_of(c * ch, 8), ch)]
      return pltpu.async_copy(tab.at[iv], bufs[b], gsem[b])

    def start_store(c, b):
      dst = pl.multiple_of(base0 + c * ch, 8)
      return pltpu.async_copy(bufs[b], out.at[pl.ds(dst, ch)], ssem[b])

    def grp(g, carry):
      c0 = g * group
      gobjs = {k: start_gather(c0 + k, k % nbuf) for k in range(ga)}
      sobjs = {}
      for j in range(group):
        gobjs.pop(j).wait()
        if j + ga < group:
          if j + ga - nbuf >= 0:
            sobjs.pop(j + ga - nbuf).wait()
          gobjs[j + ga] = start_gather(c0 + j + ga, (j + ga) % nbuf)
        sobjs[j] = start_store(c0 + j, j % nbuf)
      for j in sorted(sobjs):
        sobjs[j].wait()
      return carry

    lax.fori_loop(0, nchunks // group, grp, 0)

  return gk


def _chain_body(ic_ref, ir_ref, old_s_ref, old_t_ref, f_s_ref, f_t_ref,
                updf_s_ref, updf_t_ref):
  ic = ic_ref[...]
  ir = ir_ref[...]
  eq = ic == ir
  jj = lax.broadcasted_iota(jnp.int32, (BN, BN), 1)
  ii = lax.broadcasted_iota(jnp.int32, (BN, BN), 0)
  eqlt = eq & (jj < ii)
  pred = jnp.max(jnp.where(eqlt, jj, -1), axis=1, keepdims=True)
  depth = jnp.sum(eqlt.astype(jnp.int32), axis=1, keepdims=True)
  fin = jnp.max(jnp.where(eq & (jj >= ii), jj, -1), axis=1, keepdims=True)
  psel = (jj == pred).astype(jnp.float32)
  fsel = (jj == fin).astype(jnp.float32)
  maxd = jnp.max(depth)

  old_s = old_s_ref[...]
  old_t = old_t_ref[...]
  f_s = f_s_ref[...]
  f_t = f_t_ref[...]

  def one_bank(d, upd, old, f):
    prev = jnp.where(depth == 0, old,
                     jnp.dot(psel, upd, preferred_element_type=jnp.float32))
    v = MOM * prev + (1.0 - MOM) * f
    r = v * lax.rsqrt(jnp.sum(v * v, axis=1, keepdims=True))
    return jnp.where(depth == d, r, upd)

  def cond(carry):
    return carry[0] <= maxd

  def body(carry):
    d, us, ut = carry
    return d + 1, one_bank(d, us, old_s, f_s), one_bank(d, ut, old_t, f_t)

  zeros = jnp.zeros((BN, D), jnp.float32)
  _, upd_s, upd_t = lax.while_loop(cond, body, (0, zeros, zeros))

  updf_s_ref[...] = jnp.dot(fsel, upd_s, preferred_element_type=jnp.float32)
  updf_t_ref[...] = jnp.dot(fsel, upd_t, preferred_element_type=jnp.float32)


def _chain(idx, old_s, old_t, f_s, f_t):
  return pl.pallas_call(
      _chain_body,
      out_shape=(
          jax.ShapeDtypeStruct((BN, D), jnp.float32),
          jax.ShapeDtypeStruct((BN, D), jnp.float32),
      ),
  )(idx.reshape(BN, 1), idx.reshape(1, BN), old_s, old_t, f_s, f_t)


def _copy_body(lay_ref, in_blk, out_blk):
  del lay_ref
  out_blk[...] = in_blk[0]


def _copy_layer(memory, layer):
  lay = jnp.asarray(layer, jnp.int32).reshape(1)
  grid_spec = pltpu.PrefetchScalarGridSpec(
      num_scalar_prefetch=1,
      grid=(CAP // _COPY_ROWS,),
      in_specs=[
          pl.BlockSpec((1, _COPY_ROWS, D), lambda i, lay: (lay[0], i, 0)),
      ],
      out_specs=pl.BlockSpec((_COPY_ROWS, D), lambda i, lay: (i, 0)),
  )
  return pl.pallas_call(
      _copy_body,
      grid_spec=grid_spec,
      out_shape=jax.ShapeDtypeStruct((CAP, D), jnp.float32),
  )(lay, memory)


_RING = 16


def _patch_body(idx_ref, mem_ref, upd_ref, out_ref, sem):
  del mem_ref

  def drain():
    pltpu.make_async_copy(upd_ref.at[pl.ds(0, 1), :],
                          out_ref.at[pl.ds(0, 1), :], sem).wait()

  def step(i, carry):
    r = idx_ref[i]
    pltpu.make_async_copy(upd_ref.at[pl.ds(i, 1), :],
                          out_ref.at[pl.ds(r, 1), :], sem).start()

    @pl.when(i >= _RING)
    def _():
      drain()

    return carry

  lax.fori_loop(0, BN, step, 0)

  def fin_step(i, carry):
    drain()
    return carry

  lax.fori_loop(0, _RING, fin_step, 0)


def _patch(idx, pmem, updf):
  vmem = pl.BlockSpec(memory_space=pltpu.VMEM)
  anyspace = pl.BlockSpec(memory_space=pl.ANY)
  return pl.pallas_call(
      _patch_body,
      in_specs=[
          pl.BlockSpec(memory_space=pltpu.SMEM),
          anyspace, vmem,
      ],
      out_shape=jax.ShapeDtypeStruct((CAP, D), jnp.float32),
      out_specs=anyspace,
      scratch_shapes=[pltpu.SemaphoreType.DMA],
      input_output_aliases={1: 0},
  )(idx, pmem, updf)


def kernel(f_s, f_t, s_layer, t_layer, idx, contrast_idx, memory_v1,
           memory_v2):
  idx = idx.astype(jnp.int32)
  soff = jnp.asarray(s_layer, jnp.int32) * CAP
  toff = jnp.asarray(t_layer, jnp.int32) * CAP
  mem1f = memory_v1.reshape(4 * CAP, D)
  mem2f = memory_v2.reshape(4 * CAP, D)
  fi2 = jnp.concatenate([idx[:, None], contrast_idx.astype(jnp.int32)],
                        axis=1)
  fidx = jnp.swapaxes(fi2, 0, 1).reshape(-1)

  g_small = _make_sc_gather2(BN, 32, False)
  old_s, old_t = g_small(mem1f, mem2f, idx + soff, idx + toff)

  updf_s, updf_t = _chain(idx, old_s, old_t, f_s, f_t)

  g_big = _make_sc_gather1((KN + 1) * BN, 96, 19)
  pmem_s = _patch(idx, _copy_layer(memory_v1, s_layer), updf_s)
  w_s = g_big(pmem_s, fidx)
  pmem_t = _patch(idx, _copy_layer(memory_v2, t_layer), updf_t)
  w_t = g_big(pmem_t, fidx)
  return (jnp.swapaxes(w_s.reshape(KN + 1, BN, D), 0, 1),
          jnp.swapaxes(w_t.reshape(KN + 1, BN, D), 0, 1))


# speedup vs baseline: 32.5262x; 1.0674x over previous
import functools

import jax
import jax.numpy as jnp
from jax import lax
from jax.experimental import pallas as pl
from jax.experimental.pallas import tpu as pltpu
from jax.experimental.pallas import tpu_sc as plsc

CAP = 100000
D = 128
BN = 1024
KN = 512
MOM = 0.5

_COPY_ROWS = 4000


_NBUF = 4


def _make_sc_gather2(n_idx, ch, same_idx, group=20):
  _GROUP = group
  info = plsc.get_sparse_core_info()
  nw = info.num_cores * info.num_subcores
  n_per = n_idx // nw
  assert n_per * nw == n_idx
  assert n_per % ch == 0 and ch % 8 == 0 and ch <= 128
  nchunks = n_per // ch
  pipelined = nchunks % _GROUP == 0 and nchunks >= _GROUP

  mesh = plsc.VectorSubcoreMesh(core_axis_name="c", subcore_axis_name="s")
  scratch = [pltpu.VMEM((n_per,), jnp.int32)]
  if not same_idx:
    scratch.append(pltpu.VMEM((n_per,), jnp.int32))
  nbuf = _NBUF if pipelined else 1
  scratch += [pltpu.VMEM((ch, D), jnp.float32) for _ in range(2 * nbuf)]
  scratch += [pltpu.SemaphoreType.DMA for _ in range(4 * nbuf)]

  @functools.partial(
      pl.kernel,
      mesh=mesh,
      out_type=(
          jax.ShapeDtypeStruct((n_idx, D), jnp.float32),
          jax.ShapeDtypeStruct((n_idx, D), jnp.float32),
      ),
      scratch_types=tuple(scratch),
  )
  def gk(tab_a, tab_b, idxr_a, idxr_b, out_a, out_b, *scr):
    wid = lax.axis_index("s") * info.num_cores + lax.axis_index("c")
    base0 = wid * n_per

    pos = 0
    idx_all_a = scr[pos]
    pos += 1
    if same_idx:
      idx_all_b = idx_all_a
    else:
      idx_all_b = scr[pos]
      pos += 1
    bufs_a = scr[pos:pos + nbuf]
    bufs_b = scr[pos + nbuf:pos + 2 * nbuf]
    pos += 2 * nbuf
    gsem_a = scr[pos:pos + nbuf]
    gsem_b = scr[pos + nbuf:pos + 2 * nbuf]
    ssem_a = scr[pos + 2 * nbuf:pos + 3 * nbuf]
    ssem_b = scr[pos + 3 * nbuf:pos + 4 * nbuf]

    pltpu.sync_copy(idxr_a.at[pl.ds(base0, n_per)], idx_all_a)
    if not same_idx:
      pltpu.sync_copy(idxr_b.at[pl.ds(base0, n_per)], idx_all_b)

    def start_gather(c, b):
      iva = idx_all_a.at[pl.ds(pl.multiple_of(c * ch, 8), ch)]
      ivb = idx_all_b.at[pl.ds(pl.multiple_of(c * ch, 8), ch)]
      return (pltpu.async_copy(tab_a.at[iva], bufs_a[b], gsem_a[b]),
              pltpu.async_copy(tab_b.at[ivb], bufs_b[b], gsem_b[b]))

    def start_store(c, b):
      dst = pl.multiple_of(base0 + c * ch, 8)
      return (pltpu.async_copy(bufs_a[b], out_a.at[pl.ds(dst, ch)], ssem_a[b]),
              pltpu.async_copy(bufs_b[b], out_b.at[pl.ds(dst, ch)], ssem_b[b]))

    if not pipelined:
      def body(c, carry):
        ga, gb = start_gather(c, 0)
        ga.wait()
        gb.wait()
        sa, sb = start_store(c, 0)
        sa.wait()
        sb.wait()
        return carry

      lax.fori_loop(0, nchunks, body, 0)
      return

    def group(g, carry):
      c0 = g * _GROUP
      gobjs = {}
      sobjs = {}
      gobjs[0] = start_gather(c0 + 0, 0)
      gobjs[1] = start_gather(c0 + 1, 1)
      for j in range(_GROUP):
        ga, gb = gobjs.pop(j)
        ga.wait()
        gb.wait()
        if j + 2 < _GROUP:
          if j - 2 >= 0:
            sa, sb = sobjs.pop(j - 2)
            sa.wait()
            sb.wait()
          gobjs[j + 2] = start_gather(c0 + j + 2, (j + 2) % _NBUF)
        sobjs[j] = start_store(c0 + j, j % _NBUF)
      for j in sorted(sobjs):
        sa, sb = sobjs[j]
        sa.wait()
        sb.wait()
      return carry

    lax.fori_loop(0, nchunks // _GROUP, group, 0)

  return gk


def _make_sc_gather1(n_idx, ch, group, nbuf=6, ga=4, patch=False):
  info = plsc.get_sparse_core_info()
  nw = info.num_cores * info.num_subcores
  n_per = n_idx // nw
  assert n_per * nw == n_idx and n_per % ch == 0
  nchunks = n_per // ch
  assert nchunks % group == 0 and ga < nbuf <= group
  ppt = BN // info.num_subcores

  mesh = plsc.VectorSubcoreMesh(core_axis_name="c", subcore_axis_name="s")
  scratch = [pltpu.VMEM((n_per,), jnp.int32)]
  scratch += [pltpu.VMEM((ch, D), jnp.float32) for _ in range(nbuf)]
  scratch += [pltpu.SemaphoreType.DMA for _ in range(2 * nbuf)]
  if patch:
    scratch += [pltpu.VMEM((ppt,), jnp.int32),
                pltpu.VMEM((ppt, D), jnp.float32),
                pltpu.SemaphoreType.DMA]

  @functools.partial(
      pl.kernel,
      mesh=mesh,
      out_type=jax.ShapeDtypeStruct((n_idx, D), jnp.float32),
      scratch_types=tuple(scratch),
  )
  def gk(tab, *rest):
    if patch:
      updf, idxp, idxr, out = rest[:4]
      scr = rest[4:]
    else:
      idxr, out = rest[:2]
      scr = rest[2:]
    wid = lax.axis_index("s") * info.num_cores + lax.axis_index("c")
    base0 = wid * n_per
    idx_all = scr[0]
    bufs = scr[1:1 + nbuf]
    gsem = scr[1 + nbuf:1 + 2 * nbuf]
    ssem = scr[1 + 2 * nbuf:1 + 3 * nbuf]

    if patch:
      pidx, prow, psem = scr[1 + 3 * nbuf:1 + 3 * nbuf + 3]
      pbase = lax.axis_index("s") * ppt
      pltpu.sync_copy(idxp.at[pl.ds(pbase, ppt)], pidx)
      pltpu.sync_copy(updf.at[pl.ds(pbase, ppt)], prow)
      pltpu.async_copy(prow, tab.at[pidx], psem).wait()
      plsc.subcore_barrier()

    pltpu.sync_copy(idxr.at[pl.ds(base0, n_per)], idx_all)

    def start_gather(c, b):
      iv = idx_all.at[pl.ds(pl.multiple_of(c * ch, 8), ch)]
      return pltpu.async_copy(tab.at[iv], bufs[b], gsem[b])

    def start_store(c, b):
      dst = pl.multiple_of(base0 + c * ch, 8)
      return pltpu.async_copy(bufs[b], out.at[pl.ds(dst, ch)], ssem[b])

    def grp(g, carry):
      c0 = g * group
      gobjs = {k: start_gather(c0 + k, k % nbuf) for k in range(ga)}
      sobjs = {}
      for j in range(group):
        gobjs.pop(j).wait()
        if j + ga < group:
          if j + ga - nbuf >= 0:
            sobjs.pop(j + ga - nbuf).wait()
          gobjs[j + ga] = start_gather(c0 + j + ga, (j + ga) % nbuf)
        sobjs[j] = start_store(c0 + j, j % nbuf)
      for j in sorted(sobjs):
        sobjs[j].wait()
      return carry

    lax.fori_loop(0, nchunks // group, grp, 0)

  return gk


def _chain_body(ic_ref, ir_ref, old_s_ref, old_t_ref, f_s_ref, f_t_ref,
                updf_s_ref, updf_t_ref):
  ic = ic_ref[...]
  ir = ir_ref[...]
  eq = ic == ir
  jj = lax.broadcasted_iota(jnp.int32, (BN, BN), 1)
  ii = lax.broadcasted_iota(jnp.int32, (BN, BN), 0)
  eqlt = eq & (jj < ii)
  pred = jnp.max(jnp.where(eqlt, jj, -1), axis=1, keepdims=True)
  depth = jnp.sum(eqlt.astype(jnp.int32), axis=1, keepdims=True)
  fin = jnp.max(jnp.where(eq & (jj >= ii), jj, -1), axis=1, keepdims=True)
  psel = (jj == pred).astype(jnp.float32)
  fsel = (jj == fin).astype(jnp.float32)
  maxd = jnp.max(depth)

  old_s = old_s_ref[...]
  old_t = old_t_ref[...]
  f_s = f_s_ref[...]
  f_t = f_t_ref[...]

  def one_bank(d, upd, old, f):
    prev = jnp.where(depth == 0, old,
                     jnp.dot(psel, upd, preferred_element_type=jnp.float32))
    v = MOM * prev + (1.0 - MOM) * f
    r = v * lax.rsqrt(jnp.sum(v * v, axis=1, keepdims=True))
    return jnp.where(depth == d, r, upd)

  def cond(carry):
    return carry[0] <= maxd

  def body(carry):
    d, us, ut = carry
    return d + 1, one_bank(d, us, old_s, f_s), one_bank(d, ut, old_t, f_t)

  zeros = jnp.zeros((BN, D), jnp.float32)
  _, upd_s, upd_t = lax.while_loop(cond, body, (0, zeros, zeros))

  updf_s_ref[...] = jnp.dot(fsel, upd_s, preferred_element_type=jnp.float32)
  updf_t_ref[...] = jnp.dot(fsel, upd_t, preferred_element_type=jnp.float32)


def _chain(idx, old_s, old_t, f_s, f_t):
  return pl.pallas_call(
      _chain_body,
      out_shape=(
          jax.ShapeDtypeStruct((BN, D), jnp.float32),
          jax.ShapeDtypeStruct((BN, D), jnp.float32),
      ),
  )(idx.reshape(BN, 1), idx.reshape(1, BN), old_s, old_t, f_s, f_t)


def _copy_body(lay_ref, in_blk, out_blk):
  del lay_ref
  out_blk[...] = in_blk[0]


def _copy_layer(memory, layer):
  lay = jnp.asarray(layer, jnp.int32).reshape(1)
  grid_spec = pltpu.PrefetchScalarGridSpec(
      num_scalar_prefetch=1,
      grid=(CAP // _COPY_ROWS,),
      in_specs=[
          pl.BlockSpec((1, _COPY_ROWS, D), lambda i, lay: (lay[0], i, 0)),
      ],
      out_specs=pl.BlockSpec((_COPY_ROWS, D), lambda i, lay: (i, 0)),
  )
  return pl.pallas_call(
      _copy_body,
      grid_spec=grid_spec,
      out_shape=jax.ShapeDtypeStruct((CAP, D), jnp.float32),
  )(lay, memory)


_RING = 16


def _patch_body(idx_ref, mem_ref, upd_ref, out_ref, sem):
  del mem_ref

  def drain():
    pltpu.make_async_copy(upd_ref.at[pl.ds(0, 1), :],
                          out_ref.at[pl.ds(0, 1), :], sem).wait()

  def step(i, carry):
    r = idx_ref[i]
    pltpu.make_async_copy(upd_ref.at[pl.ds(i, 1), :],
                          out_ref.at[pl.ds(r, 1), :], sem).start()

    @pl.when(i >= _RING)
    def _():
      drain()

    return carry

  lax.fori_loop(0, BN, step, 0)

  def fin_step(i, carry):
    drain()
    return carry

  lax.fori_loop(0, _RING, fin_step, 0)


def _patch(idx, pmem, updf):
  vmem = pl.BlockSpec(memory_space=pltpu.VMEM)
  anyspace = pl.BlockSpec(memory_space=pl.ANY)
  return pl.pallas_call(
      _patch_body,
      in_specs=[
          pl.BlockSpec(memory_space=pltpu.SMEM),
          anyspace, vmem,
      ],
      out_shape=jax.ShapeDtypeStruct((CAP, D), jnp.float32),
      out_specs=anyspace,
      scratch_shapes=[pltpu.SemaphoreType.DMA],
      input_output_aliases={1: 0},
  )(idx, pmem, updf)


def kernel(f_s, f_t, s_layer, t_layer, idx, contrast_idx, memory_v1,
           memory_v2):
  idx = idx.astype(jnp.int32)
  soff = jnp.asarray(s_layer, jnp.int32) * CAP
  toff = jnp.asarray(t_layer, jnp.int32) * CAP
  mem1f = memory_v1.reshape(4 * CAP, D)
  mem2f = memory_v2.reshape(4 * CAP, D)
  fi2 = jnp.concatenate([idx[:, None], contrast_idx.astype(jnp.int32)],
                        axis=1)
  fidx = jnp.swapaxes(fi2, 0, 1).reshape(-1)

  g_small = _make_sc_gather2(BN, 32, False)
  old_s, old_t = g_small(mem1f, mem2f, idx + soff, idx + toff)

  updf_s, updf_t = _chain(idx, old_s, old_t, f_s, f_t)

  g_big = _make_sc_gather1((KN + 1) * BN, 96, 19, patch=True)
  cref_s = jax.new_ref(_copy_layer(memory_v1, s_layer))
  w_s = g_big(cref_s, updf_s, idx, fidx)
  cref_t = jax.new_ref(_copy_layer(memory_v2, t_layer))
  w_t = g_big(cref_t, updf_t, idx, fidx)
  return (jnp.swapaxes(w_s.reshape(KN + 1, BN, D), 0, 1),
          jnp.swapaxes(w_t.reshape(KN + 1, BN, D), 0, 1))
